# Initial kernel scaffold; baseline (speedup 1.0000x reference)
import jax
import jax.numpy as jnp
from jax.experimental import pallas as pl


def kernel(x, edge_index, gradint_dir, select_index, id2id, new_rows, cols, negs, id_negs, W1, W2, W_std):
    raise NotImplementedError("write your pallas kernel here")


# speedup vs baseline: 4.1496x; 4.1496x over previous
import functools

import jax
import jax.numpy as jnp
from jax import lax
from jax.experimental import pallas as pl
from jax.experimental.pallas import tpu as pltpu
from jax.experimental.pallas import tpu_sc as plsc

N = 10000
D = 128
NE = 320000
S = 4096
P = 8192
NEG = 10
TEMP = 0.07
AUG_W = 1e-05
INS_W = 1e-05

NC, NS, L = 2, 16, 16
NW = NC * NS
K = 128
ECH = 80
EW = ECH * K
PAD_E = NW * EW
ACC_ROWS = 10240
ROWS_PER_TILE = ACC_ROWS // NS
TRASH = N

def _mesh():
    return plsc.VectorSubcoreMesh(
        core_axis_name="c", subcore_axis_name="s", num_cores=NC, num_subcores=NS)


def _ring(n, nb, issue, consume):
    for b in range(min(nb, n)):
        issue(b, b)
    nfull = (n // nb) * nb
    if nfull > 0:
        @pl.loop(0, nfull, step=nb)
        def _(j):
            for b in range(nb):
                cur = j + b

                def body(cur=cur, b=b):
                    consume(cur, b)
                    nxt = cur + nb

                    @pl.when(nxt < n)
                    def _():
                        issue(nxt, b)

                body()
    for r in range(nfull, n):
        consume(r, r % nb)


def _spmm_body(pre, src_i, dst_i, out, sbuf, dst_v, rb0, rb1, acc,
               rsem0, rsem1, isem0, isem1, isem2, isem3):
    cid = lax.axis_index("c")
    sid = lax.axis_index("s")
    wid = cid * NS + sid
    pltpu.sync_copy(dst_i.at[wid], dst_v)

    @pl.loop(0, K)
    def _(r):
        for c in range(D // L):
            rb0[r, pl.ds(c * L, L)] = jnp.zeros((L,), jnp.float32)

    nfull = ROWS_PER_TILE // K
    for c in range(nfull):
        pltpu.sync_copy(rb0, acc.at[pl.ds(sid * ROWS_PER_TILE + c * K, K)])
    rem = ROWS_PER_TILE - nfull * K
    if rem:
        pltpu.sync_copy(rb0.at[pl.ds(0, rem)],
                        acc.at[pl.ds(sid * ROWS_PER_TILE + nfull * K, rem)])
    plsc.subcore_barrier()

    rbufs = (rb0, rb1)
    rsems = (rsem0, rsem1)
    isems = (isem0, isem1, isem2, isem3)

    def issue_idx(cur, s):
        pltpu.async_copy(src_i.at[wid, cur], sbuf.at[s], isems[s])

    def wait_idx(s):
        pltpu.make_async_copy(src_i.at[wid, 0], sbuf.at[s], isems[s]).wait()

    def issue_rows(s_idx, b):
        pltpu.async_copy(pre.at[sbuf.at[s_idx]], rbufs[b], rsems[b])

    def wait_rows(s_idx, b):
        pltpu.make_async_copy(pre.at[sbuf.at[s_idx]], rbufs[b],
                              rsems[b]).wait()

    for c in range(4):
        issue_idx(c, c)
    for c in range(2):
        wait_idx(c)
        issue_rows(c, c)

    @pl.loop(0, ECH, step=4)
    def _(j):
        for t in range(4):
            cur = j + t
            p2 = t % 2
            wait_rows(t, p2)

            @pl.when(cur + 4 < ECH)
            def _(cur=cur, t=t):
                issue_idx(cur + 4, t)

            pltpu.sync_copy(rbufs[p2], acc.at[dst_v.at[cur]], add=True)

            @pl.when(cur + 2 < ECH)
            def _(t=t, p2=p2):
                wait_idx((t + 2) % 4)
                issue_rows((t + 2) % 4, p2)

    plsc.subcore_barrier()
    pltpu.sync_copy(acc.at[pl.ds(sid * ROWS_PER_TILE, ROWS_PER_TILE)],
                    out.at[cid, pl.ds(sid * ROWS_PER_TILE, ROWS_PER_TILE)])


def _spmm(pre, src_i, dst_i):
    return pl.kernel(
        _spmm_body,
        out_type=jax.ShapeDtypeStruct((NC, ACC_ROWS, D), jnp.float32),
        mesh=_mesh(),
        scratch_types=[
            pltpu.VMEM((4, K), jnp.int32),
            pltpu.VMEM((ECH, K), jnp.int32),
            pltpu.VMEM((K, D), jnp.float32),
            pltpu.VMEM((K, D), jnp.float32),
            pltpu.VMEM_SHARED((ACC_ROWS, D), jnp.float32),
            pltpu.SemaphoreType.DMA,
            pltpu.SemaphoreType.DMA,
            pltpu.SemaphoreType.DMA,
            pltpu.SemaphoreType.DMA,
            pltpu.SemaphoreType.DMA,
            pltpu.SemaphoreType.DMA,
        ],
    )(pre, src_i, dst_i)

SEG_CH = (1, 2, 2, 20, 10)
GD_CH = 2


def _gather_body(h, gdir, sel_i, row_i, col_i, neg_i, idn_i, gd_i,
                 o_sel, o_row, o_col, o_neg, o_idn, o_gd,
                 idx_v, rb0, rb1, sem0, sem1):
    cid = lax.axis_index("c")
    sid = lax.axis_index("s")
    wid = cid * NS + sid
    rbufs = (rb0, rb1)
    sems = (sem0, sem1)

    segs = ((sel_i, o_sel, SEG_CH[0], h),
            (row_i, o_row, SEG_CH[1], h),
            (col_i, o_col, SEG_CH[2], h),
            (neg_i, o_neg, SEG_CH[3], h),
            (idn_i, o_idn, SEG_CH[4], h),
            (gd_i, o_gd, GD_CH, gdir))

    off = 0
    for seg_i, seg_o, nch, table in segs:
        pltpu.sync_copy(seg_i.at[wid], idx_v.at[pl.ds(off, nch)])

        def issue(cur, b, table=table, off=off):
            pltpu.async_copy(table.at[idx_v.at[off + cur]], rbufs[b], sems[b])

        def consume(cur, b, table=table, off=off, seg_o=seg_o, nch=nch):
            pltpu.make_async_copy(table.at[idx_v.at[off + cur]],
                                  rbufs[b], sems[b]).wait()
            pltpu.sync_copy(rbufs[b],
                            seg_o.at[pl.ds(wid * nch * K + cur * K, K)])

        _ring(nch, 2, issue, consume)
        off += nch


_TOTAL_CH = sum(SEG_CH) + GD_CH

def _gather(h, gdir, sel_i, row_i, col_i, neg_i, idn_i, gd_i):
    return pl.kernel(
        _gather_body,
        out_type=[
            jax.ShapeDtypeStruct((S, D), jnp.float32),
            jax.ShapeDtypeStruct((P, D), jnp.float32),
            jax.ShapeDtypeStruct((P, D), jnp.float32),
            jax.ShapeDtypeStruct((P * NEG, D), jnp.float32),
            jax.ShapeDtypeStruct((S * NEG, D), jnp.float32),
            jax.ShapeDtypeStruct((P, D), jnp.float32),
        ],
        mesh=_mesh(),
        scratch_types=[
            pltpu.VMEM((_TOTAL_CH, K), jnp.int32),
            pltpu.VMEM((K, D), jnp.float32),
            pltpu.VMEM((K, D), jnp.float32),
            pltpu.SemaphoreType.DMA,
            pltpu.SemaphoreType.DMA,
        ],
    )(h, gdir, sel_i, row_i, col_i, neg_i, idn_i, gd_i)


_MM_B = 1000


def _mm_body(x_ref, w_ref, o_ref):
    o_ref[...] = jnp.dot(x_ref[...], w_ref[...],
                         preferred_element_type=jnp.float32)


def _mm(x, w):
    return pl.pallas_call(
        _mm_body,
        grid=(N // _MM_B,),
        in_specs=[pl.BlockSpec((_MM_B, D), lambda i: (i, 0)),
                  pl.BlockSpec((D, D), lambda i: (0, 0))],
        out_specs=pl.BlockSpec((_MM_B, D), lambda i: (i, 0)),
        out_shape=jax.ShapeDtypeStruct((N, D), jnp.float32),
    )(x, w)


def _relu_mm_body(p_ref, w_ref, o_ref):
    s = jnp.maximum(p_ref[0] + p_ref[1], 0.0)
    o_ref[...] = jnp.dot(s, w_ref[...], preferred_element_type=jnp.float32)


def _relu_mm(parts, w):
    return pl.pallas_call(
        _relu_mm_body,
        grid=(N // _MM_B,),
        in_specs=[pl.BlockSpec((NC, _MM_B, D), lambda i: (0, i, 0)),
                  pl.BlockSpec((D, D), lambda i: (0, 0))],
        out_specs=pl.BlockSpec((_MM_B, D), lambda i: (i, 0)),
        out_shape=jax.ShapeDtypeStruct((N, D), jnp.float32),
    )(parts, w)


def _relu_add_body(p_ref, o_ref):
    o_ref[...] = jnp.maximum(p_ref[0] + p_ref[1], 0.0)


def _relu_add(parts):
    return pl.pallas_call(
        _relu_add_body,
        grid=(N // _MM_B,),
        in_specs=[pl.BlockSpec((NC, _MM_B, D), lambda i: (0, i, 0))],
        out_specs=pl.BlockSpec((_MM_B, D), lambda i: (i, 0)),
        out_shape=jax.ShapeDtypeStruct((N, D), jnp.float32),
    )(parts)


def _softplus(x):
    return jnp.maximum(x, 0.0) + jnp.log1p(jnp.exp(-jnp.abs(x)))


_LB = 1024


def _loss_edges_body(row_ref, col_ref, neg_ref, gd_ref, w_ref, og_ref, oa_ref):
    row = row_ref[...]
    col = col_ref[...]
    gdir = gd_ref[...]
    wstd = w_ref[...]
    pos = jnp.sum(row * col, axis=1)
    std = jax.nn.sigmoid(jnp.sum(row * wstd, axis=1))
    nrm = jnp.sqrt(jnp.sum(gdir * gdir, axis=1, keepdims=True))
    gd = (gdir / jnp.maximum(nrm, 1e-12)) * std[:, None]
    dpos = jnp.sum(gd * col, axis=1)
    gae = 10.0 * _softplus(-pos)
    aug = 10.0 * _softplus(-(pos + dpos))
    for j in range(NEG):
        nh = neg_ref[:, j * D:(j + 1) * D]
        nl = jnp.sum(row * nh, axis=1)
        dnl = jnp.sum(gd * nh, axis=1)
        gae = gae + _softplus(nl)
        aug = aug + _softplus(nl + dnl)

    @pl.when(pl.program_id(0) == 0)
    def _():
        og_ref[...] = jnp.zeros_like(og_ref)
        oa_ref[...] = jnp.zeros_like(oa_ref)

    og_ref[...] = og_ref[...] + jnp.sum(gae)
    oa_ref[...] = oa_ref[...] + jnp.sum(aug)


def _loss_edges(row_h, col_h, neg2d, gdir_rows, wstd_t):
    return pl.pallas_call(
        _loss_edges_body,
        grid=(P // _LB,),
        in_specs=[pl.BlockSpec((_LB, D), lambda i: (i, 0)),
                  pl.BlockSpec((_LB, D), lambda i: (i, 0)),
                  pl.BlockSpec((_LB, NEG * D), lambda i: (i, 0)),
                  pl.BlockSpec((_LB, D), lambda i: (i, 0)),
                  pl.BlockSpec((1, D), lambda i: (0, 0))],
        out_specs=[pl.BlockSpec((1, 1), lambda i: (0, 0)),
                   pl.BlockSpec((1, 1), lambda i: (0, 0))],
        out_shape=[jax.ShapeDtypeStruct((1, 1), jnp.float32),
                   jax.ShapeDtypeStruct((1, 1), jnp.float32)],
    )(row_h, col_h, neg2d, gdir_rows, wstd_t)


def _loss_inst_body(gae_ref, gdir_ref, neg_ref, w_ref, o_ref):
    gae_h = gae_ref[...]
    gdir = gdir_ref[...]
    wstd = w_ref[...]
    std = jax.nn.sigmoid(jnp.sum(gae_h * wstd, axis=1))
    nrm = jnp.sqrt(jnp.sum(gdir * gdir, axis=1, keepdims=True))
    aug = gae_h + (gdir / jnp.maximum(nrm, 1e-12)) * std[:, None]
    pos = jnp.sum(aug * gae_h, axis=1) / TEMP
    acc = _softplus(-pos)
    for j in range(NEG):
        nh = neg_ref[:, j * D:(j + 1) * D]
        acc = acc + _softplus(jnp.sum(aug * nh, axis=1) / TEMP)

    @pl.when(pl.program_id(0) == 0)
    def _():
        o_ref[...] = jnp.zeros_like(o_ref)

    o_ref[...] = o_ref[...] + jnp.sum(acc)


def _loss_inst(gae_h, gdir, idneg2d, wstd_t):
    return pl.pallas_call(
        _loss_inst_body,
        grid=(S // _LB,),
        in_specs=[pl.BlockSpec((_LB, D), lambda i: (i, 0)),
                  pl.BlockSpec((_LB, D), lambda i: (i, 0)),
                  pl.BlockSpec((_LB, NEG * D), lambda i: (i, 0)),
                  pl.BlockSpec((1, D), lambda i: (0, 0))],
        out_specs=pl.BlockSpec((1, 1), lambda i: (0, 0)),
        out_shape=jax.ShapeDtypeStruct((1, 1), jnp.float32),
    )(gae_h, gdir, idneg2d, wstd_t)


def kernel(x, edge_index, gradint_dir, select_index, id2id, new_rows, cols,
           negs, id_negs, W1, W2, W_std):
    src = edge_index[0]
    dst = edge_index[1]
    pad = PAD_E - NE
    src_p = jnp.concatenate([src, jnp.zeros((pad,), jnp.int32)]).reshape(NW, ECH, K)
    dst_p = jnp.concatenate([dst, jnp.full((pad,), TRASH, jnp.int32)]).reshape(NW, ECH, K)

    pre1 = _mm(x, W1)
    parts1 = _spmm(pre1, src_p, dst_p)
    pre2 = _relu_mm(parts1, W2)
    parts2 = _spmm(pre2, src_p, dst_p)
    h = _relu_add(parts2)

    row_idx = jnp.take(select_index, new_rows)
    sel_i = select_index.reshape(NW, SEG_CH[0], K)
    row_i = row_idx.reshape(NW, SEG_CH[1], K)
    col_i = cols.reshape(NW, SEG_CH[2], K)
    neg_i = negs.reshape(NW, SEG_CH[3], K)
    idn_i = id_negs.reshape(NW, SEG_CH[4], K)
    gd_i = new_rows.reshape(NW, GD_CH, K)

    gae_h, row_h, col_h, neg_f, idn_f, gdir_rows = _gather(
        h, gradint_dir, sel_i, row_i, col_i, neg_i, idn_i, gd_i)

    neg2d = neg_f.reshape(P, NEG * D)
    idn2d = idn_f.reshape(S, NEG * D)
    wstd_t = W_std.reshape(1, D)

    lg, la = _loss_edges(row_h, col_h, neg2d, gdir_rows, wstd_t)
    l2 = _loss_inst(gae_h, gradint_dir, idn2d, wstd_t)

    gae_loss = lg[0, 0] / P
    aug_loss = la[0, 0] / P
    inst_loss = l2[0, 0] / S
    return gae_loss + AUG_W * aug_loss + INS_W * inst_loss


# speedup vs baseline: 8.2428x; 1.9864x over previous
import functools

import jax
import jax.numpy as jnp
from jax import lax
from jax.experimental import pallas as pl
from jax.experimental.pallas import tpu as pltpu
from jax.experimental.pallas import tpu_sc as plsc

N = 10000
D = 128
NE = 320000
S = 4096
P = 8192
NEG = 10
TEMP = 0.07
AUG_W = 1e-05
INS_W = 1e-05

NC, NS, L = 2, 16, 16
NW = NC * NS
K = 128
EK = 64
ECH = 160
EW = ECH * EK
PAD_E = NW * EW
ACC_ROWS = 10240
ROWS_PER_TILE = ACC_ROWS // NS
TRASH = N

def _mesh():
    return plsc.VectorSubcoreMesh(
        core_axis_name="c", subcore_axis_name="s", num_cores=NC, num_subcores=NS)


def _spmm_body(pre, src_i, dst_i, out, sbuf, dbuf, rb0, rb1, rb2, rb3, acc,
               rsem0, rsem1, rsem2, rsem3, isem0, isem1, isem2, isem3,
               ssem0, ssem1, ssem2, ssem3, dsem0, dsem1, dsem2, dsem3):
    cid = lax.axis_index("c")
    sid = lax.axis_index("s")
    wid = cid * NS + sid

    @pl.loop(0, EK)
    def _(r):
        for c in range(D // L):
            rb0[r, pl.ds(c * L, L)] = jnp.zeros((L,), jnp.float32)

    for c in range(ROWS_PER_TILE // EK):
        pltpu.sync_copy(rb0, acc.at[pl.ds(sid * ROWS_PER_TILE + c * EK, EK)])
    plsc.subcore_barrier()

    rbufs = (rb0, rb1, rb2, rb3)
    rsems = (rsem0, rsem1, rsem2, rsem3)
    isems = (isem0, isem1, isem2, isem3)
    ssems = (ssem0, ssem1, ssem2, ssem3)
    dsems = (dsem0, dsem1, dsem2, dsem3)

    def issue_idx(cur, s):
        pltpu.async_copy(src_i.at[wid, cur], sbuf.at[s], isems[s])

    def wait_idx(s):
        pltpu.make_async_copy(src_i.at[wid, 0], sbuf.at[s], isems[s]).wait()

    def issue_didx(cur, s):
        pltpu.async_copy(dst_i.at[wid, cur], dbuf.at[s], dsems[s])

    def wait_didx(s):
        pltpu.make_async_copy(dst_i.at[wid, 0], dbuf.at[s], dsems[s]).wait()

    def issue_rows(s, b):
        pltpu.async_copy(pre.at[sbuf.at[s]], rbufs[b], rsems[b])

    def wait_rows(b):
        pltpu.make_async_copy(pre.at[sbuf.at[b]], rbufs[b], rsems[b]).wait()

    def issue_scat(b):
        pltpu.async_copy(rbufs[b], acc.at[dbuf.at[b]], ssems[b], add=True)

    def wait_scat(b):
        pltpu.make_async_copy(rbufs[b], acc.at[dbuf.at[b]], ssems[b]).wait()

    for c in range(4):
        issue_idx(c, c)
        issue_didx(c, c)
    for c in range(2):
        wait_idx(c)
        issue_rows(c, c)

    @pl.loop(0, ECH, step=4)
    def _(j):
        for t in range(4):
            cur = j + t
            wait_rows(t)

            @pl.when(cur + 4 < ECH)
            def _(cur=cur, t=t):
                issue_idx(cur + 4, t)

            wait_didx(t)
            issue_scat(t)
            t2 = (t + 2) % 4

            @pl.when(cur + 2 < ECH)
            def _(cur=cur, t2=t2):
                @pl.when(cur >= 2)
                def _(cur=cur, t2=t2):
                    wait_scat(t2)
                    issue_didx(cur + 2, t2)

                wait_idx(t2)
                issue_rows(t2, t2)

    for c in range(ECH - 4, ECH):
        wait_scat(c % 4)
    plsc.subcore_barrier()
    pltpu.sync_copy(acc.at[pl.ds(sid * ROWS_PER_TILE, ROWS_PER_TILE)],
                    out.at[cid, pl.ds(sid * ROWS_PER_TILE, ROWS_PER_TILE)])


def _spmm(pre, src_i, dst_i):
    return pl.kernel(
        _spmm_body,
        out_type=jax.ShapeDtypeStruct((NC, ACC_ROWS, D), jnp.float32),
        mesh=_mesh(),
        scratch_types=[
            pltpu.VMEM((4, EK), jnp.int32),
            pltpu.VMEM((4, EK), jnp.int32),
            pltpu.VMEM((EK, D), jnp.float32),
            pltpu.VMEM((EK, D), jnp.float32),
            pltpu.VMEM((EK, D), jnp.float32),
            pltpu.VMEM((EK, D), jnp.float32),
            pltpu.VMEM_SHARED((ACC_ROWS, D), jnp.float32),
        ] + [pltpu.SemaphoreType.DMA] * 16,
    )(pre, src_i, dst_i)

SEG_CH = (1, 2, 2, 20, 10)
GD_CH = 2


def _gather_body(h, gdir, sel_i, row_i, col_i, neg_i, idn_i, gd_i,
                 o_sel, o_row, o_col, o_neg, o_idn, o_gd,
                 idx_v, rb0, rb1, rb2, rb3,
                 rsem0, rsem1, rsem2, rsem3, wsem0, wsem1, wsem2, wsem3):
    cid = lax.axis_index("c")
    sid = lax.axis_index("s")
    wid = cid * NS + sid
    rbufs = (rb0, rb1, rb2, rb3)
    rsems = (rsem0, rsem1, rsem2, rsem3)
    wsems = (wsem0, wsem1, wsem2, wsem3)

    segs = ((sel_i, o_sel, SEG_CH[0], h),
            (row_i, o_row, SEG_CH[1], h),
            (col_i, o_col, SEG_CH[2], h),
            (neg_i, o_neg, SEG_CH[3], h),
            (idn_i, o_idn, SEG_CH[4], h),
            (gd_i, o_gd, GD_CH, gdir))

    off = 0
    for seg_i, seg_o, nch, table in segs:
        pltpu.sync_copy(seg_i.at[wid], idx_v.at[pl.ds(off, nch)])

        def issue(cur, b, table=table, off=off):
            pltpu.async_copy(table.at[idx_v.at[off + cur]], rbufs[b],
                             rsems[b])

        def wait_r(b, table=table, off=off):
            pltpu.make_async_copy(table.at[idx_v.at[off]], rbufs[b],
                                  rsems[b]).wait()

        def write(cur, b, seg_o=seg_o, nch=nch):
            pltpu.async_copy(rbufs[b],
                             seg_o.at[pl.ds(wid * nch * K + cur * K, K)],
                             wsems[b])

        def wait_w(b, seg_o=seg_o):
            pltpu.make_async_copy(rbufs[b], seg_o.at[pl.ds(0, K)],
                                  wsems[b]).wait()

        for c in range(min(2, nch)):
            issue(c, c)
        nfull = (nch // 4) * 4
        if nfull:
            @pl.loop(0, nfull, step=4)
            def _(j, issue=issue, wait_r=wait_r, write=write, wait_w=wait_w,
                  nch=nch):
                for t in range(4):
                    cur = j + t
                    wait_r(t)
                    write(cur, t)
                    t2 = (t + 2) % 4

                    @pl.when(cur + 2 < nch)
                    def _(cur=cur, t2=t2, issue=issue, wait_w=wait_w):
                        @pl.when(cur >= 2)
                        def _():
                            wait_w(t2)

                        issue(cur + 2, t2)

        for cur in range(nfull, nch):
            b = cur % 4
            wait_r(b)
            write(cur, b)
            if cur + 2 < nch:
                if cur >= 2:
                    wait_w((cur + 2) % 4)
                issue(cur + 2, (cur + 2) % 4)
        for c in range(max(0, nch - 4), nch):
            wait_w(c % 4)
        off += nch


_TOTAL_CH = sum(SEG_CH) + GD_CH

def _gather(h, gdir, sel_i, row_i, col_i, neg_i, idn_i, gd_i):
    return pl.kernel(
        _gather_body,
        out_type=[
            jax.ShapeDtypeStruct((S, D), jnp.float32),
            jax.ShapeDtypeStruct((P, D), jnp.float32),
            jax.ShapeDtypeStruct((P, D), jnp.float32),
            jax.ShapeDtypeStruct((P * NEG, D), jnp.float32),
            jax.ShapeDtypeStruct((S * NEG, D), jnp.float32),
            jax.ShapeDtypeStruct((P, D), jnp.float32),
        ],
        mesh=_mesh(),
        scratch_types=[
            pltpu.VMEM((_TOTAL_CH, K), jnp.int32),
            pltpu.VMEM((K, D), jnp.float32),
            pltpu.VMEM((K, D), jnp.float32),
            pltpu.VMEM((K, D), jnp.float32),
            pltpu.VMEM((K, D), jnp.float32),
        ] + [pltpu.SemaphoreType.DMA] * 8,
    )(h, gdir, sel_i, row_i, col_i, neg_i, idn_i, gd_i)


_MM_B = 1000


def _mm_body(x_ref, w_ref, o_ref):
    o_ref[...] = jnp.dot(x_ref[...], w_ref[...],
                         preferred_element_type=jnp.float32)


def _mm(x, w):
    return pl.pallas_call(
        _mm_body,
        grid=(N // _MM_B,),
        in_specs=[pl.BlockSpec((_MM_B, D), lambda i: (i, 0)),
                  pl.BlockSpec((D, D), lambda i: (0, 0))],
        out_specs=pl.BlockSpec((_MM_B, D), lambda i: (i, 0)),
        out_shape=jax.ShapeDtypeStruct((N, D), jnp.float32),
    )(x, w)


def _relu_mm_body(p_ref, w_ref, o_ref):
    s = jnp.maximum(p_ref[0] + p_ref[1], 0.0)
    o_ref[...] = jnp.dot(s, w_ref[...], preferred_element_type=jnp.float32)


def _relu_mm(parts, w):
    return pl.pallas_call(
        _relu_mm_body,
        grid=(N // _MM_B,),
        in_specs=[pl.BlockSpec((NC, _MM_B, D), lambda i: (0, i, 0)),
                  pl.BlockSpec((D, D), lambda i: (0, 0))],
        out_specs=pl.BlockSpec((_MM_B, D), lambda i: (i, 0)),
        out_shape=jax.ShapeDtypeStruct((N, D), jnp.float32),
    )(parts, w)


def _relu_add_body(p_ref, o_ref):
    o_ref[...] = jnp.maximum(p_ref[0] + p_ref[1], 0.0)


def _relu_add(parts):
    return pl.pallas_call(
        _relu_add_body,
        grid=(N // _MM_B,),
        in_specs=[pl.BlockSpec((NC, _MM_B, D), lambda i: (0, i, 0))],
        out_specs=pl.BlockSpec((_MM_B, D), lambda i: (i, 0)),
        out_shape=jax.ShapeDtypeStruct((N, D), jnp.float32),
    )(parts)


def _softplus(x):
    return jnp.maximum(x, 0.0) + jnp.log1p(jnp.exp(-jnp.abs(x)))


_LB = 1024


def _loss_edges_body(row_ref, col_ref, neg_ref, gd_ref, w_ref, og_ref, oa_ref):
    row = row_ref[...]
    col = col_ref[...]
    gdir = gd_ref[...]
    wstd = w_ref[...]
    pos = jnp.sum(row * col, axis=1)
    std = jax.nn.sigmoid(jnp.sum(row * wstd, axis=1))
    nrm = jnp.sqrt(jnp.sum(gdir * gdir, axis=1, keepdims=True))
    gd = (gdir / jnp.maximum(nrm, 1e-12)) * std[:, None]
    dpos = jnp.sum(gd * col, axis=1)
    gae = 10.0 * _softplus(-pos)
    aug = 10.0 * _softplus(-(pos + dpos))
    for j in range(NEG):
        nh = neg_ref[:, j * D:(j + 1) * D]
        nl = jnp.sum(row * nh, axis=1)
        dnl = jnp.sum(gd * nh, axis=1)
        gae = gae + _softplus(nl)
        aug = aug + _softplus(nl + dnl)

    @pl.when(pl.program_id(0) == 0)
    def _():
        og_ref[...] = jnp.zeros_like(og_ref)
        oa_ref[...] = jnp.zeros_like(oa_ref)

    og_ref[...] = og_ref[...] + jnp.sum(gae)
    oa_ref[...] = oa_ref[...] + jnp.sum(aug)


def _loss_edges(row_h, col_h, neg2d, gdir_rows, wstd_t):
    return pl.pallas_call(
        _loss_edges_body,
        grid=(P // _LB,),
        in_specs=[pl.BlockSpec((_LB, D), lambda i: (i, 0)),
                  pl.BlockSpec((_LB, D), lambda i: (i, 0)),
                  pl.BlockSpec((_LB, NEG * D), lambda i: (i, 0)),
                  pl.BlockSpec((_LB, D), lambda i: (i, 0)),
                  pl.BlockSpec((1, D), lambda i: (0, 0))],
        out_specs=[pl.BlockSpec((1, 1), lambda i: (0, 0)),
                   pl.BlockSpec((1, 1), lambda i: (0, 0))],
        out_shape=[jax.ShapeDtypeStruct((1, 1), jnp.float32),
                   jax.ShapeDtypeStruct((1, 1), jnp.float32)],
    )(row_h, col_h, neg2d, gdir_rows, wstd_t)


def _loss_inst_body(gae_ref, gdir_ref, neg_ref, w_ref, o_ref):
    gae_h = gae_ref[...]
    gdir = gdir_ref[...]
    wstd = w_ref[...]
    std = jax.nn.sigmoid(jnp.sum(gae_h * wstd, axis=1))
    nrm = jnp.sqrt(jnp.sum(gdir * gdir, axis=1, keepdims=True))
    aug = gae_h + (gdir / jnp.maximum(nrm, 1e-12)) * std[:, None]
    pos = jnp.sum(aug * gae_h, axis=1) / TEMP
    acc = _softplus(-pos)
    for j in range(NEG):
        nh = neg_ref[:, j * D:(j + 1) * D]
        acc = acc + _softplus(jnp.sum(aug * nh, axis=1) / TEMP)

    @pl.when(pl.program_id(0) == 0)
    def _():
        o_ref[...] = jnp.zeros_like(o_ref)

    o_ref[...] = o_ref[...] + jnp.sum(acc)


def _loss_inst(gae_h, gdir, idneg2d, wstd_t):
    return pl.pallas_call(
        _loss_inst_body,
        grid=(S // _LB,),
        in_specs=[pl.BlockSpec((_LB, D), lambda i: (i, 0)),
                  pl.BlockSpec((_LB, D), lambda i: (i, 0)),
                  pl.BlockSpec((_LB, NEG * D), lambda i: (i, 0)),
                  pl.BlockSpec((1, D), lambda i: (0, 0))],
        out_specs=pl.BlockSpec((1, 1), lambda i: (0, 0)),
        out_shape=jax.ShapeDtypeStruct((1, 1), jnp.float32),
    )(gae_h, gdir, idneg2d, wstd_t)


def kernel(x, edge_index, gradint_dir, select_index, id2id, new_rows, cols,
           negs, id_negs, W1, W2, W_std):
    src = edge_index[0]
    dst = edge_index[1]
    pad = PAD_E - NE
    pad_i = jnp.arange(pad, dtype=jnp.int32)
    src_p = jnp.concatenate([src, pad_i % N]).reshape(NW, ECH, EK)
    dst_p = jnp.concatenate([dst, TRASH + pad_i % (ACC_ROWS - N)]).reshape(NW, ECH, EK)

    pre1 = _mm(x, W1)
    parts1 = _spmm(pre1, src_p, dst_p)
    pre2 = _relu_mm(parts1, W2)
    parts2 = _spmm(pre2, src_p, dst_p)
    h = _relu_add(parts2)

    row_idx = jnp.take(select_index, new_rows)
    sel_i = select_index.reshape(NW, SEG_CH[0], K)
    row_i = row_idx.reshape(NW, SEG_CH[1], K)
    col_i = cols.reshape(NW, SEG_CH[2], K)
    neg_i = negs.reshape(NW, SEG_CH[3], K)
    idn_i = id_negs.reshape(NW, SEG_CH[4], K)
    gd_i = new_rows.reshape(NW, GD_CH, K)

    gae_h, row_h, col_h, neg_f, idn_f, gdir_rows = _gather(
        h, gradint_dir, sel_i, row_i, col_i, neg_i, idn_i, gd_i)

    neg2d = neg_f.reshape(P, NEG * D)
    idn2d = idn_f.reshape(S, NEG * D)
    wstd_t = W_std.reshape(1, D)

    lg, la = _loss_edges(row_h, col_h, neg2d, gdir_rows, wstd_t)
    l2 = _loss_inst(gae_h, gradint_dir, idn2d, wstd_t)

    gae_loss = lg[0, 0] / P
    aug_loss = la[0, 0] / P
    inst_loss = l2[0, 0] / S
    return gae_loss + AUG_W * aug_loss + INS_W * inst_loss


# speedup vs baseline: 9.3126x; 1.1298x over previous
import functools

import jax
import jax.numpy as jnp
from jax import lax
from jax.experimental import pallas as pl
from jax.experimental.pallas import tpu as pltpu
from jax.experimental.pallas import tpu_sc as plsc

N = 10000
D = 128
NE = 320000
S = 4096
P = 8192
NEG = 10
TEMP = 0.07
AUG_W = 1e-05
INS_W = 1e-05

NC, NS, L = 2, 16, 16
NW = NC * NS
K = 128
EK = 64
ECH = 160
EW = ECH * EK
PAD_E = NW * EW
ACC_ROWS = 10240
ROWS_PER_TILE = ACC_ROWS // NS
TRASH = N

def _mesh():
    return plsc.VectorSubcoreMesh(
        core_axis_name="c", subcore_axis_name="s", num_cores=NC, num_subcores=NS)


def _spmm_body(pre, src_i, dst_i, out, sbuf, dbuf, rb0, rb1, rb2, rb3, acc,
               rsem0, rsem1, rsem2, rsem3, isem0, isem1, isem2, isem3,
               ssem0, ssem1, ssem2, ssem3, dsem0, dsem1, dsem2, dsem3):
    cid = lax.axis_index("c")
    sid = lax.axis_index("s")
    wid = cid * NS + sid

    @pl.loop(0, EK)
    def _(r):
        for c in range(D // L):
            rb0[r, pl.ds(c * L, L)] = jnp.zeros((L,), jnp.float32)

    for c in range(ROWS_PER_TILE // EK):
        pltpu.sync_copy(rb0, acc.at[pl.ds(sid * ROWS_PER_TILE + c * EK, EK)])
    plsc.subcore_barrier()

    rbufs = (rb0, rb1, rb2, rb3)
    rsems = (rsem0, rsem1, rsem2, rsem3)
    isems = (isem0, isem1, isem2, isem3)
    ssems = (ssem0, ssem1, ssem2, ssem3)
    dsems = (dsem0, dsem1, dsem2, dsem3)

    def issue_idx(cur, s):
        pltpu.async_copy(src_i.at[wid, cur], sbuf.at[s], isems[s])

    def wait_idx(s):
        pltpu.make_async_copy(src_i.at[wid, 0], sbuf.at[s], isems[s]).wait()

    def issue_didx(cur, s):
        pltpu.async_copy(dst_i.at[wid, cur], dbuf.at[s], dsems[s])

    def wait_didx(s):
        pltpu.make_async_copy(dst_i.at[wid, 0], dbuf.at[s], dsems[s]).wait()

    def issue_rows(s, b):
        pltpu.async_copy(pre.at[sbuf.at[s]], rbufs[b], rsems[b])

    def wait_rows(b):
        pltpu.make_async_copy(pre.at[sbuf.at[b]], rbufs[b], rsems[b]).wait()

    def issue_scat(b):
        pltpu.async_copy(rbufs[b], acc.at[dbuf.at[b]], ssems[b], add=True)

    def wait_scat(b):
        pltpu.make_async_copy(rbufs[b], acc.at[dbuf.at[b]], ssems[b]).wait()

    for c in range(4):
        issue_idx(c, c)
        issue_didx(c, c)
    for c in range(2):
        wait_idx(c)
        issue_rows(c, c)

    @pl.loop(0, ECH, step=4)
    def _(j):
        for t in range(4):
            cur = j + t
            wait_rows(t)

            @pl.when(cur + 4 < ECH)
            def _(cur=cur, t=t):
                issue_idx(cur + 4, t)

            wait_didx(t)
            issue_scat(t)
            t2 = (t + 2) % 4

            @pl.when(cur + 2 < ECH)
            def _(cur=cur, t2=t2):
                @pl.when(cur >= 2)
                def _(cur=cur, t2=t2):
                    wait_scat(t2)
                    issue_didx(cur + 2, t2)

                wait_idx(t2)
                issue_rows(t2, t2)

    for c in range(ECH - 4, ECH):
        wait_scat(c % 4)
    plsc.subcore_barrier()
    pltpu.sync_copy(acc.at[pl.ds(sid * ROWS_PER_TILE, ROWS_PER_TILE)],
                    out.at[cid, pl.ds(sid * ROWS_PER_TILE, ROWS_PER_TILE)])


def _spmm(pre, src_i, dst_i):
    return pl.kernel(
        _spmm_body,
        out_type=jax.ShapeDtypeStruct((NC, ACC_ROWS, D), jnp.float32),
        mesh=_mesh(),
        scratch_types=[
            pltpu.VMEM((4, EK), jnp.int32),
            pltpu.VMEM((4, EK), jnp.int32),
            pltpu.VMEM((EK, D), jnp.float32),
            pltpu.VMEM((EK, D), jnp.float32),
            pltpu.VMEM((EK, D), jnp.float32),
            pltpu.VMEM((EK, D), jnp.float32),
            pltpu.VMEM_SHARED((ACC_ROWS, D), jnp.float32),
        ] + [pltpu.SemaphoreType.DMA] * 16,
    )(pre, src_i, dst_i)

_BARRIER = "barrier-marker"
SEG_CH = (1, 2, 2, 20, 10)
GD_CH = 2


def _gather_body(h, gdir, sel2, col_i, neg_i, idn_i, gd_i,
                 o_sel, o_row, o_col, o_neg, o_idn, o_gd,
                 idx_v, rb0, rb1, rb2, rb3,
                 rsem0, rsem1, rsem2, rsem3, wsem0, wsem1, wsem2, wsem3):
    cid = lax.axis_index("c")
    sid = lax.axis_index("s")
    wid = cid * NS + sid
    rbufs = (rb0, rb1, rb2, rb3)
    rsems = (rsem0, rsem1, rsem2, rsem3)
    wsems = (wsem0, wsem1, wsem2, wsem3)

    for c2 in range(2):
        pltpu.sync_copy(sel2.at[sid * 2 + c2], idx_v.at[34 + c2])
        pltpu.async_copy(h.at[idx_v.at[34 + c2]], rbufs[c2], rsems[c2])
    for c2 in range(2):
        pltpu.make_async_copy(h.at[idx_v.at[34 + c2]], rbufs[c2],
                              rsems[c2]).wait()
        pltpu.sync_copy(rbufs[c2], o_sel.at[pl.ds((sid * 2 + c2) * K, K)])

    segs = ((col_i, o_col, SEG_CH[2], h),
            (neg_i, o_neg, SEG_CH[3], h),
            (idn_i, o_idn, SEG_CH[4], h),
            (gd_i, o_gd, GD_CH, gdir),
            (_BARRIER, o_row, SEG_CH[1], o_sel))

    off = 0
    for seg_i, seg_o, nch, table in segs:
        if seg_i is _BARRIER:
            plsc.subcore_barrier()
            seg_i = None
            off -= GD_CH
        if seg_i is not None:
            pltpu.sync_copy(seg_i.at[wid], idx_v.at[pl.ds(off, nch)])

        def issue(cur, b, table=table, off=off):
            pltpu.async_copy(table.at[idx_v.at[off + cur]], rbufs[b],
                             rsems[b])

        def wait_r(b, table=table, off=off):
            pltpu.make_async_copy(table.at[idx_v.at[off]], rbufs[b],
                                  rsems[b]).wait()

        def write(cur, b, seg_o=seg_o, nch=nch):
            pltpu.async_copy(rbufs[b],
                             seg_o.at[pl.ds(wid * nch * K + cur * K, K)],
                             wsems[b])

        def wait_w(b, seg_o=seg_o):
            pltpu.make_async_copy(rbufs[b], seg_o.at[pl.ds(0, K)],
                                  wsems[b]).wait()

        for c in range(min(2, nch)):
            issue(c, c)
        nfull = (nch // 4) * 4
        if nfull:
            @pl.loop(0, nfull, step=4)
            def _(j, issue=issue, wait_r=wait_r, write=write, wait_w=wait_w,
                  nch=nch):
                for t in range(4):
                    cur = j + t
                    wait_r(t)
                    write(cur, t)
                    t2 = (t + 2) % 4

                    @pl.when(cur + 2 < nch)
                    def _(cur=cur, t2=t2, issue=issue, wait_w=wait_w):
                        @pl.when(cur >= 2)
                        def _():
                            wait_w(t2)

                        issue(cur + 2, t2)

        for cur in range(nfull, nch):
            b = cur % 4
            wait_r(b)
            write(cur, b)
            if cur + 2 < nch:
                if cur >= 2:
                    wait_w((cur + 2) % 4)
                issue(cur + 2, (cur + 2) % 4)
        for c in range(max(0, nch - 4), nch):
            wait_w(c % 4)
        off += nch


_TOTAL_CH = 36

def _gather(h, gdir, sel2, col_i, neg_i, idn_i, gd_i):
    return pl.kernel(
        _gather_body,
        out_type=[
            jax.ShapeDtypeStruct((S, D), jnp.float32),
            jax.ShapeDtypeStruct((P, D), jnp.float32),
            jax.ShapeDtypeStruct((P, D), jnp.float32),
            jax.ShapeDtypeStruct((P * NEG, D), jnp.float32),
            jax.ShapeDtypeStruct((S * NEG, D), jnp.float32),
            jax.ShapeDtypeStruct((P, D), jnp.float32),
        ],
        mesh=_mesh(),
        scratch_types=[
            pltpu.VMEM((_TOTAL_CH, K), jnp.int32),
            pltpu.VMEM((K, D), jnp.float32),
            pltpu.VMEM((K, D), jnp.float32),
            pltpu.VMEM((K, D), jnp.float32),
            pltpu.VMEM((K, D), jnp.float32),
        ] + [pltpu.SemaphoreType.DMA] * 8,
    )(h, gdir, sel2, col_i, neg_i, idn_i, gd_i)


_MM_B = 1000


def _mm_body(x_ref, w_ref, o_ref):
    o_ref[...] = jnp.dot(x_ref[...], w_ref[...],
                         preferred_element_type=jnp.float32)


def _mm(x, w):
    return pl.pallas_call(
        _mm_body,
        grid=(N // _MM_B,),
        in_specs=[pl.BlockSpec((_MM_B, D), lambda i: (i, 0)),
                  pl.BlockSpec((D, D), lambda i: (0, 0))],
        out_specs=pl.BlockSpec((_MM_B, D), lambda i: (i, 0)),
        out_shape=jax.ShapeDtypeStruct((N, D), jnp.float32),
    )(x, w)


def _relu_mm_body(p_ref, w_ref, o_ref):
    s = jnp.maximum(p_ref[0] + p_ref[1], 0.0)
    o_ref[...] = jnp.dot(s, w_ref[...], preferred_element_type=jnp.float32)


def _relu_mm(parts, w):
    return pl.pallas_call(
        _relu_mm_body,
        grid=(N // _MM_B,),
        in_specs=[pl.BlockSpec((NC, _MM_B, D), lambda i: (0, i, 0)),
                  pl.BlockSpec((D, D), lambda i: (0, 0))],
        out_specs=pl.BlockSpec((_MM_B, D), lambda i: (i, 0)),
        out_shape=jax.ShapeDtypeStruct((N, D), jnp.float32),
    )(parts, w)


def _relu_add_body(p_ref, o_ref):
    o_ref[...] = jnp.maximum(p_ref[0] + p_ref[1], 0.0)


def _relu_add(parts):
    return pl.pallas_call(
        _relu_add_body,
        grid=(N // _MM_B,),
        in_specs=[pl.BlockSpec((NC, _MM_B, D), lambda i: (0, i, 0))],
        out_specs=pl.BlockSpec((_MM_B, D), lambda i: (i, 0)),
        out_shape=jax.ShapeDtypeStruct((N, D), jnp.float32),
    )(parts)


def _softplus(x):
    return jnp.maximum(x, 0.0) + jnp.log1p(jnp.exp(-jnp.abs(x)))


_LB = 1024


def _loss_edges_body(row_ref, col_ref, neg_ref, gd_ref, w_ref, og_ref, oa_ref):
    ones = jnp.full((D, 1), 1.0, jnp.float32)
    row = row_ref[...]
    col = col_ref[...]
    gdir = gd_ref[...]
    wstd = w_ref[...]

    def rdot(a, b):
        return jnp.dot(a * b, ones, preferred_element_type=jnp.float32)

    pos = rdot(row, col)
    std = jax.nn.sigmoid(rdot(row, jnp.broadcast_to(wstd, row.shape)))
    gn = gdir / jnp.maximum(jnp.sqrt(rdot(gdir, gdir)), 1e-12)
    gae = 10.0 * _softplus(-pos)
    aug = 10.0 * _softplus(-(pos + std * rdot(gn, col)))
    for j in range(NEG):
        nh = neg_ref[j]
        nl = rdot(row, nh)
        gae = gae + _softplus(nl)
        aug = aug + _softplus(nl + std * rdot(gn, nh))

    @pl.when(pl.program_id(0) == 0)
    def _():
        og_ref[...] = jnp.zeros_like(og_ref)
        oa_ref[...] = jnp.zeros_like(oa_ref)

    og_ref[...] = og_ref[...] + jnp.sum(gae)
    oa_ref[...] = oa_ref[...] + jnp.sum(aug)


def _loss_edges(row_h, col_h, neg3, gdir_rows, wstd_t):
    return pl.pallas_call(
        _loss_edges_body,
        grid=(P // _LB,),
        in_specs=[pl.BlockSpec((_LB, D), lambda i: (i, 0)),
                  pl.BlockSpec((_LB, D), lambda i: (i, 0)),
                  pl.BlockSpec((NEG, _LB, D), lambda i: (0, i, 0)),
                  pl.BlockSpec((_LB, D), lambda i: (i, 0)),
                  pl.BlockSpec((1, D), lambda i: (0, 0))],
        out_specs=[pl.BlockSpec((1, 1), lambda i: (0, 0)),
                   pl.BlockSpec((1, 1), lambda i: (0, 0))],
        out_shape=[jax.ShapeDtypeStruct((1, 1), jnp.float32),
                   jax.ShapeDtypeStruct((1, 1), jnp.float32)],
    )(row_h, col_h, neg3, gdir_rows, wstd_t)


def _loss_inst_body(gae_ref, gdir_ref, neg_ref, w_ref, o_ref):
    ones = jnp.full((D, 1), 1.0, jnp.float32)
    gae_h = gae_ref[...]
    gdir = gdir_ref[...]
    wstd = w_ref[...]

    def rdot(a, b):
        return jnp.dot(a * b, ones, preferred_element_type=jnp.float32)

    std = jax.nn.sigmoid(rdot(gae_h, jnp.broadcast_to(wstd, gae_h.shape)))
    gn = gdir / jnp.maximum(jnp.sqrt(rdot(gdir, gdir)), 1e-12)
    pos = (rdot(gae_h, gae_h) + std * rdot(gn, gae_h)) / TEMP
    acc = _softplus(-pos)
    for j in range(NEG):
        nh = neg_ref[j]
        nl = (rdot(gae_h, nh) + std * rdot(gn, nh)) / TEMP
        acc = acc + _softplus(nl)

    @pl.when(pl.program_id(0) == 0)
    def _():
        o_ref[...] = jnp.zeros_like(o_ref)

    o_ref[...] = o_ref[...] + jnp.sum(acc)


def _loss_inst(gae_h, gdir, idn3, wstd_t):
    return pl.pallas_call(
        _loss_inst_body,
        grid=(S // _LB,),
        in_specs=[pl.BlockSpec((_LB, D), lambda i: (i, 0)),
                  pl.BlockSpec((_LB, D), lambda i: (i, 0)),
                  pl.BlockSpec((NEG, _LB, D), lambda i: (0, i, 0)),
                  pl.BlockSpec((1, D), lambda i: (0, 0))],
        out_specs=pl.BlockSpec((1, 1), lambda i: (0, 0)),
        out_shape=jax.ShapeDtypeStruct((1, 1), jnp.float32),
    )(gae_h, gdir, idn3, wstd_t)


def kernel(x, edge_index, gradint_dir, select_index, id2id, new_rows, cols,
           negs, id_negs, W1, W2, W_std):
    src = edge_index[0]
    dst = edge_index[1]
    pad = PAD_E - NE
    pad_i = jnp.arange(pad, dtype=jnp.int32)
    src_p = jnp.concatenate([src, pad_i % N]).reshape(NW, ECH, EK)
    dst_p = jnp.concatenate([dst, TRASH + pad_i % (ACC_ROWS - N)]).reshape(NW, ECH, EK)

    pre1 = _mm(x, W1)
    parts1 = _spmm(pre1, src_p, dst_p)
    pre2 = _relu_mm(parts1, W2)
    parts2 = _spmm(pre2, src_p, dst_p)
    h = _relu_add(parts2)

    sel_i = select_index.reshape(NW, K)
    col_i = cols.reshape(NW, SEG_CH[2], K)
    neg_i = negs.T.reshape(NW, SEG_CH[3], K)
    idn_i = id_negs.T.reshape(NW, SEG_CH[4], K)
    gd_i = new_rows.reshape(NW, GD_CH, K)

    gae_h, row_h, col_h, neg_f, idn_f, gdir_rows = _gather(
        h, gradint_dir, sel_i, col_i, neg_i, idn_i, gd_i)

    neg3 = neg_f.reshape(NEG, P, D)
    idn3 = idn_f.reshape(NEG, S, D)
    wstd_t = W_std.reshape(1, D)

    lg, la = _loss_edges(row_h, col_h, neg3, gdir_rows, wstd_t)
    l2 = _loss_inst(gae_h, gradint_dir, idn3, wstd_t)

    gae_loss = lg[0, 0] / P
    aug_loss = la[0, 0] / P
    inst_loss = l2[0, 0] / S
    return gae_loss + AUG_W * aug_loss + INS_W * inst_loss


# speedup vs baseline: 10.4076x; 1.1176x over previous
import functools

import jax
import jax.numpy as jnp
from jax import lax
from jax.experimental import pallas as pl
from jax.experimental.pallas import tpu as pltpu
from jax.experimental.pallas import tpu_sc as plsc

N = 10000
D = 128
NE = 320000
S = 4096
P = 8192
NEG = 10
TEMP = 0.07
AUG_W = 1e-05
INS_W = 1e-05

NC, NS, L = 2, 16, 16
NW = NC * NS
K = 128
EK = 64
ECH = 160
EW = ECH * EK
PAD_E = NW * EW
ACC_ROWS = 10240
ROWS_PER_TILE = ACC_ROWS // NS
TRASH = N

def _mesh():
    return plsc.VectorSubcoreMesh(
        core_axis_name="c", subcore_axis_name="s", num_cores=NC, num_subcores=NS)


def _spmm_body(pre, src_i, dst_i, out, sbuf, dbuf, rb0, rb1, rb2, rb3, acc,
               rsem0, rsem1, rsem2, rsem3, isem0, isem1, isem2, isem3,
               ssem0, ssem1, ssem2, ssem3, dsem0, dsem1, dsem2, dsem3):
    cid = lax.axis_index("c")
    sid = lax.axis_index("s")
    wid = cid * NS + sid

    @pl.loop(0, EK)
    def _(r):
        for c in range(D // L):
            rb0[r, pl.ds(c * L, L)] = jnp.zeros((L,), jnp.float32)

    for c in range(ROWS_PER_TILE // EK):
        pltpu.sync_copy(rb0, acc.at[pl.ds(sid * ROWS_PER_TILE + c * EK, EK)])
    plsc.subcore_barrier()

    rbufs = (rb0, rb1, rb2, rb3)
    rsems = (rsem0, rsem1, rsem2, rsem3)
    isems = (isem0, isem1, isem2, isem3)
    ssems = (ssem0, ssem1, ssem2, ssem3)
    dsems = (dsem0, dsem1, dsem2, dsem3)

    def issue_idx(cur, s):
        pltpu.async_copy(src_i.at[wid, cur], sbuf.at[s], isems[s])

    def wait_idx(s):
        pltpu.make_async_copy(src_i.at[wid, 0], sbuf.at[s], isems[s]).wait()

    def issue_didx(cur, s):
        pltpu.async_copy(dst_i.at[wid, cur], dbuf.at[s], dsems[s])

    def wait_didx(s):
        pltpu.make_async_copy(dst_i.at[wid, 0], dbuf.at[s], dsems[s]).wait()

    def issue_rows(s, b):
        pltpu.async_copy(pre.at[sbuf.at[s]], rbufs[b], rsems[b])

    def wait_rows(b):
        pltpu.make_async_copy(pre.at[sbuf.at[b]], rbufs[b], rsems[b]).wait()

    def issue_scat(b):
        pltpu.async_copy(rbufs[b], acc.at[dbuf.at[b]], ssems[b], add=True)

    def wait_scat(b):
        pltpu.make_async_copy(rbufs[b], acc.at[dbuf.at[b]], ssems[b]).wait()

    for c in range(4):
        issue_idx(c, c)
        issue_didx(c, c)
    for c in range(2):
        wait_idx(c)
        issue_rows(c, c)

    @pl.loop(0, ECH, step=4)
    def _(j):
        for t in range(4):
            cur = j + t
            wait_rows(t)

            @pl.when(cur + 4 < ECH)
            def _(cur=cur, t=t):
                issue_idx(cur + 4, t)

            wait_didx(t)
            issue_scat(t)
            t2 = (t + 2) % 4

            @pl.when(cur + 2 < ECH)
            def _(cur=cur, t2=t2):
                @pl.when(cur >= 2)
                def _(cur=cur, t2=t2):
                    wait_scat(t2)
                    issue_didx(cur + 2, t2)

                wait_idx(t2)
                issue_rows(t2, t2)

    for c in range(ECH - 4, ECH):
        wait_scat(c % 4)
    plsc.subcore_barrier()
    pltpu.sync_copy(acc.at[pl.ds(sid * ROWS_PER_TILE, ROWS_PER_TILE)],
                    out.at[cid, pl.ds(sid * ROWS_PER_TILE, ROWS_PER_TILE)])


def _spmm(pre, src_i, dst_i):
    return pl.kernel(
        _spmm_body,
        out_type=jax.ShapeDtypeStruct((NC, ACC_ROWS, D), jnp.float32),
        mesh=_mesh(),
        scratch_types=[
            pltpu.VMEM((4, EK), jnp.int32),
            pltpu.VMEM((4, EK), jnp.int32),
            pltpu.VMEM((EK, D), jnp.float32),
            pltpu.VMEM((EK, D), jnp.float32),
            pltpu.VMEM((EK, D), jnp.float32),
            pltpu.VMEM((EK, D), jnp.float32),
            pltpu.VMEM_SHARED((ACC_ROWS, D), jnp.float32),
        ] + [pltpu.SemaphoreType.DMA] * 16,
    )(pre, src_i, dst_i)

_BARRIER = "barrier-marker"
SEG_CH = (1, 2, 2, 20, 10)
GD_CH = 2


def _gather_body(h, gdir, sel2, col_i, neg_i, idn_i, gd_i,
                 o_sel, o_row, o_col, o_neg, o_idn, o_gd,
                 idx_v, rb0, rb1, rb2, rb3,
                 rsem0, rsem1, rsem2, rsem3, wsem0, wsem1, wsem2, wsem3):
    cid = lax.axis_index("c")
    sid = lax.axis_index("s")
    wid = cid * NS + sid
    rbufs = (rb0, rb1, rb2, rb3)
    rsems = (rsem0, rsem1, rsem2, rsem3)
    wsems = (wsem0, wsem1, wsem2, wsem3)

    for c2 in range(2):
        pltpu.sync_copy(sel2.at[sid * 2 + c2], idx_v.at[34 + c2])
        pltpu.async_copy(h.at[idx_v.at[34 + c2]], rbufs[c2], rsems[c2])
    for c2 in range(2):
        pltpu.make_async_copy(h.at[idx_v.at[34 + c2]], rbufs[c2],
                              rsems[c2]).wait()
        pltpu.sync_copy(rbufs[c2], o_sel.at[pl.ds((sid * 2 + c2) * K, K)])

    segs = ((col_i, o_col, SEG_CH[2], h),
            (neg_i, o_neg, SEG_CH[3], h),
            (idn_i, o_idn, SEG_CH[4], h),
            (gd_i, o_gd, GD_CH, gdir),
            (_BARRIER, o_row, SEG_CH[1], o_sel))

    off = 0
    for seg_i, seg_o, nch, table in segs:
        if seg_i is _BARRIER:
            plsc.subcore_barrier()
            seg_i = None
            off -= GD_CH
        if seg_i is not None:
            pltpu.sync_copy(seg_i.at[wid], idx_v.at[pl.ds(off, nch)])

        def issue(cur, b, table=table, off=off):
            pltpu.async_copy(table.at[idx_v.at[off + cur]], rbufs[b],
                             rsems[b])

        def wait_r(b, table=table, off=off):
            pltpu.make_async_copy(table.at[idx_v.at[off]], rbufs[b],
                                  rsems[b]).wait()

        def write(cur, b, seg_o=seg_o, nch=nch):
            pltpu.async_copy(rbufs[b],
                             seg_o.at[pl.ds(wid * nch * K + cur * K, K)],
                             wsems[b])

        def wait_w(b, seg_o=seg_o):
            pltpu.make_async_copy(rbufs[b], seg_o.at[pl.ds(0, K)],
                                  wsems[b]).wait()

        for c in range(min(2, nch)):
            issue(c, c)
        nfull = (nch // 4) * 4
        if nfull:
            @pl.loop(0, nfull, step=4)
            def _(j, issue=issue, wait_r=wait_r, write=write, wait_w=wait_w,
                  nch=nch):
                for t in range(4):
                    cur = j + t
                    wait_r(t)
                    write(cur, t)
                    t2 = (t + 2) % 4

                    @pl.when(cur + 2 < nch)
                    def _(cur=cur, t2=t2, issue=issue, wait_w=wait_w):
                        @pl.when(cur >= 2)
                        def _():
                            wait_w(t2)

                        issue(cur + 2, t2)

        for cur in range(nfull, nch):
            b = cur % 4
            wait_r(b)
            write(cur, b)
            if cur + 2 < nch:
                if cur >= 2:
                    wait_w((cur + 2) % 4)
                issue(cur + 2, (cur + 2) % 4)
        for c in range(max(0, nch - 4), nch):
            wait_w(c % 4)
        off += nch


_TOTAL_CH = 36

def _gather(h, gdir, sel2, col_i, neg_i, idn_i, gd_i):
    return pl.kernel(
        _gather_body,
        out_type=[
            jax.ShapeDtypeStruct((S, D), jnp.float32),
            jax.ShapeDtypeStruct((P, D), jnp.float32),
            jax.ShapeDtypeStruct((P, D), jnp.float32),
            jax.ShapeDtypeStruct((P * NEG, D), jnp.float32),
            jax.ShapeDtypeStruct((S * NEG, D), jnp.float32),
            jax.ShapeDtypeStruct((P, D), jnp.float32),
        ],
        mesh=_mesh(),
        scratch_types=[
            pltpu.VMEM((_TOTAL_CH, K), jnp.int32),
            pltpu.VMEM((K, D), jnp.float32),
            pltpu.VMEM((K, D), jnp.float32),
            pltpu.VMEM((K, D), jnp.float32),
            pltpu.VMEM((K, D), jnp.float32),
        ] + [pltpu.SemaphoreType.DMA] * 8,
    )(h, gdir, sel2, col_i, neg_i, idn_i, gd_i)


_MM_B = 1000


def _mm_body(x_ref, w_ref, o_ref):
    o_ref[...] = jnp.dot(x_ref[...], w_ref[...],
                         preferred_element_type=jnp.float32)


def _mm(x, w):
    return pl.pallas_call(
        _mm_body,
        grid=(N // _MM_B,),
        in_specs=[pl.BlockSpec((_MM_B, D), lambda i: (i, 0)),
                  pl.BlockSpec((D, D), lambda i: (0, 0))],
        out_specs=pl.BlockSpec((_MM_B, D), lambda i: (i, 0)),
        out_shape=jax.ShapeDtypeStruct((N, D), jnp.float32),
    )(x, w)


def _relu_mm_body(p_ref, w_ref, o_ref):
    s = jnp.maximum(p_ref[0] + p_ref[1], 0.0)
    o_ref[...] = jnp.dot(s, w_ref[...], preferred_element_type=jnp.float32)


def _relu_mm(parts, w):
    return pl.pallas_call(
        _relu_mm_body,
        grid=(N // _MM_B,),
        in_specs=[pl.BlockSpec((NC, _MM_B, D), lambda i: (0, i, 0)),
                  pl.BlockSpec((D, D), lambda i: (0, 0))],
        out_specs=pl.BlockSpec((_MM_B, D), lambda i: (i, 0)),
        out_shape=jax.ShapeDtypeStruct((N, D), jnp.float32),
    )(parts, w)


def _relu_add_body(p_ref, o_ref):
    o_ref[...] = jnp.maximum(p_ref[0] + p_ref[1], 0.0)


def _relu_add(parts):
    return pl.pallas_call(
        _relu_add_body,
        grid=(N // _MM_B,),
        in_specs=[pl.BlockSpec((NC, _MM_B, D), lambda i: (0, i, 0))],
        out_specs=pl.BlockSpec((_MM_B, D), lambda i: (i, 0)),
        out_shape=jax.ShapeDtypeStruct((N, D), jnp.float32),
    )(parts)


def _softplus(x):
    return jnp.maximum(x, 0.0) + jnp.log1p(jnp.exp(-jnp.abs(x)))


_LB = 1024


def _lane_consts():
    lane = lax.broadcasted_iota(jnp.int32, (1, 128), 1)
    w = jnp.where(lane == 0, 10.0, jnp.where(lane < 11, 1.0, 0.0))
    sgn = jnp.where(lane == 0, -1.0, jnp.where(lane < 11, 1.0, 0.0))
    return w, sgn


def _packed_logits(a, other, neg_ref, extra, bd):
    prods = [a * other] + [a * neg_ref[j] for j in range(NEG)] + [a * extra]
    x = jnp.concatenate(prods, axis=1)
    return jnp.dot(x, bd, preferred_element_type=jnp.float32)


def _loss_edges_body(row_ref, col_ref, neg_ref, gd_ref, w_ref, bd_ref,
                     og_ref, oa_ref):
    row = row_ref[...]
    col = col_ref[...]
    gdir = gd_ref[...]
    wstd = jnp.broadcast_to(w_ref[...], row.shape)
    bd = bd_ref[...]
    w, sgn = _lane_consts()

    p1 = _packed_logits(row, col, neg_ref, wstd, bd)
    p2 = _packed_logits(gdir, col, neg_ref, gdir, bd)
    std = jax.nn.sigmoid(p1[:, 11:12])
    c = std / jnp.maximum(jnp.sqrt(p2[:, 11:12]), 1e-12)
    base = p1 * sgn
    aug = (p1 + c * p2) * sgn
    gae_sum = jnp.sum(w * _softplus(base))
    aug_sum = jnp.sum(w * _softplus(aug))

    @pl.when(pl.program_id(0) == 0)
    def _():
        og_ref[...] = jnp.zeros_like(og_ref)
        oa_ref[...] = jnp.zeros_like(oa_ref)

    og_ref[...] = og_ref[...] + gae_sum
    oa_ref[...] = oa_ref[...] + aug_sum


def _loss_edges(row_h, col_h, neg3, gdir_rows, wstd_t, bd):
    return pl.pallas_call(
        _loss_edges_body,
        grid=(P // _LB,),
        in_specs=[pl.BlockSpec((_LB, D), lambda i: (i, 0)),
                  pl.BlockSpec((_LB, D), lambda i: (i, 0)),
                  pl.BlockSpec((NEG, _LB, D), lambda i: (0, i, 0)),
                  pl.BlockSpec((_LB, D), lambda i: (i, 0)),
                  pl.BlockSpec((1, D), lambda i: (0, 0)),
                  pl.BlockSpec((12 * D, 128), lambda i: (0, 0))],
        out_specs=[pl.BlockSpec((1, 1), lambda i: (0, 0)),
                   pl.BlockSpec((1, 1), lambda i: (0, 0))],
        out_shape=[jax.ShapeDtypeStruct((1, 1), jnp.float32),
                   jax.ShapeDtypeStruct((1, 1), jnp.float32)],
    )(row_h, col_h, neg3, gdir_rows, wstd_t, bd)


def _loss_inst_body(gae_ref, gdir_ref, neg_ref, w_ref, bd_ref, o_ref):
    gae_h = gae_ref[...]
    gdir = gdir_ref[...]
    wstd = jnp.broadcast_to(w_ref[...], gae_h.shape)
    bd = bd_ref[...]
    w, sgn = _lane_consts()
    w = jnp.where(w > 0.0, 1.0, 0.0)

    p1 = _packed_logits(gae_h, gae_h, neg_ref, wstd, bd)
    p2 = _packed_logits(gdir, gae_h, neg_ref, gdir, bd)
    std = jax.nn.sigmoid(p1[:, 11:12])
    c = std / jnp.maximum(jnp.sqrt(p2[:, 11:12]), 1e-12)
    logits = ((p1 + c * p2) * (1.0 / TEMP)) * sgn
    acc = jnp.sum(w * _softplus(logits))

    @pl.when(pl.program_id(0) == 0)
    def _():
        o_ref[...] = jnp.zeros_like(o_ref)

    o_ref[...] = o_ref[...] + acc


def _loss_inst(gae_h, gdir, idn3, wstd_t, bd):
    return pl.pallas_call(
        _loss_inst_body,
        grid=(S // _LB,),
        in_specs=[pl.BlockSpec((_LB, D), lambda i: (i, 0)),
                  pl.BlockSpec((_LB, D), lambda i: (i, 0)),
                  pl.BlockSpec((NEG, _LB, D), lambda i: (0, i, 0)),
                  pl.BlockSpec((1, D), lambda i: (0, 0)),
                  pl.BlockSpec((12 * D, 128), lambda i: (0, 0))],
        out_specs=pl.BlockSpec((1, 1), lambda i: (0, 0)),
        out_shape=jax.ShapeDtypeStruct((1, 1), jnp.float32),
    )(gae_h, gdir, idn3, wstd_t, bd)


def kernel(x, edge_index, gradint_dir, select_index, id2id, new_rows, cols,
           negs, id_negs, W1, W2, W_std):
    src = edge_index[0]
    dst = edge_index[1]
    pad = PAD_E - NE
    pad_i = jnp.arange(pad, dtype=jnp.int32)
    src_p = jnp.concatenate([src, pad_i % N]).reshape(NW, ECH, EK)
    dst_p = jnp.concatenate([dst, TRASH + pad_i % (ACC_ROWS - N)]).reshape(NW, ECH, EK)

    pre1 = _mm(x, W1)
    parts1 = _spmm(pre1, src_p, dst_p)
    pre2 = _relu_mm(parts1, W2)
    parts2 = _spmm(pre2, src_p, dst_p)
    h = _relu_add(parts2)

    sel_i = select_index.reshape(NW, K)
    col_i = cols.reshape(NW, SEG_CH[2], K)
    neg_i = negs.T.reshape(NW, SEG_CH[3], K)
    idn_i = id_negs.T.reshape(NW, SEG_CH[4], K)
    gd_i = new_rows.reshape(NW, GD_CH, K)

    gae_h, row_h, col_h, neg_f, idn_f, gdir_rows = _gather(
        h, gradint_dir, sel_i, col_i, neg_i, idn_i, gd_i)

    neg3 = neg_f.reshape(NEG, P, D)
    idn3 = idn_f.reshape(NEG, S, D)
    wstd_t = W_std.reshape(1, D)
    bd = jnp.repeat(jnp.eye(12, 128, dtype=jnp.float32), D, axis=0)

    lg, la = _loss_edges(row_h, col_h, neg3, gdir_rows, wstd_t, bd)
    l2 = _loss_inst(gae_h, gradint_dir, idn3, wstd_t, bd)

    gae_loss = lg[0, 0] / P
    aug_loss = la[0, 0] / P
    inst_loss = l2[0, 0] / S
    return gae_loss + AUG_W * aug_loss + INS_W * inst_loss


# speedup vs baseline: 10.4315x; 1.0023x over previous
import functools

import jax
import jax.numpy as jnp
from jax import lax
from jax.experimental import pallas as pl
from jax.experimental.pallas import tpu as pltpu
from jax.experimental.pallas import tpu_sc as plsc

N = 10000
D = 128
NE = 320000
S = 4096
P = 8192
NEG = 10
TEMP = 0.07
AUG_W = 1e-05
INS_W = 1e-05

NC, NS, L = 2, 16, 16
NW = NC * NS
K = 128
EK = 64
ECH = 160
EW = ECH * EK
PAD_E = NW * EW
ACC_ROWS = 10240
ROWS_PER_TILE = ACC_ROWS // NS
TRASH = N

def _mesh():
    return plsc.VectorSubcoreMesh(
        core_axis_name="c", subcore_axis_name="s", num_cores=NC, num_subcores=NS)


def _spmm_body(pre, src_i, dst_i, out, sbuf, dbuf, rb0, rb1, rb2, rb3, acc,
               rsem0, rsem1, rsem2, rsem3, isem0, isem1, isem2, isem3,
               ssem0, ssem1, ssem2, ssem3, dsem0, dsem1, dsem2, dsem3):
    cid = lax.axis_index("c")
    sid = lax.axis_index("s")
    wid = cid * NS + sid
    rbufs = (rb0, rb1, rb2, rb3)
    rsems = (rsem0, rsem1, rsem2, rsem3)
    isems = (isem0, isem1, isem2, isem3)
    ssems = (ssem0, ssem1, ssem2, ssem3)
    dsems = (dsem0, dsem1, dsem2, dsem3)

    def issue_idx(cur, s):
        pltpu.async_copy(src_i.at[wid, cur], sbuf.at[s], isems[s])

    def wait_idx(s):
        pltpu.make_async_copy(src_i.at[wid, 0], sbuf.at[s], isems[s]).wait()

    def issue_didx(cur, s):
        pltpu.async_copy(dst_i.at[wid, cur], dbuf.at[s], dsems[s])

    def wait_didx(s):
        pltpu.make_async_copy(dst_i.at[wid, 0], dbuf.at[s], dsems[s]).wait()

    def issue_rows(s, b):
        pltpu.async_copy(pre.at[sbuf.at[s]], rbufs[b], rsems[b])

    def wait_rows(b):
        pltpu.make_async_copy(pre.at[sbuf.at[b]], rbufs[b], rsems[b]).wait()

    def issue_scat(b):
        pltpu.async_copy(rbufs[b], acc.at[dbuf.at[b]], ssems[b], add=True)

    def wait_scat(b):
        pltpu.make_async_copy(rbufs[b], acc.at[dbuf.at[b]], ssems[b]).wait()

    for c in range(4):
        issue_idx(c, c)
        issue_didx(c, c)
    for c in range(2):
        wait_idx(c)
        issue_rows(c, c)

    @pl.loop(0, EK)
    def _(r):
        for c in range(D // L):
            rb3[r, pl.ds(c * L, L)] = jnp.zeros((L,), jnp.float32)

    for c in range(ROWS_PER_TILE // EK):
        pltpu.sync_copy(rb3, acc.at[pl.ds(sid * ROWS_PER_TILE + c * EK, EK)])
    plsc.subcore_barrier()

    @pl.loop(0, ECH, step=4)
    def _(j):
        for t in range(4):
            cur = j + t
            wait_rows(t)

            @pl.when(cur + 4 < ECH)
            def _(cur=cur, t=t):
                issue_idx(cur + 4, t)

            wait_didx(t)
            issue_scat(t)
            t2 = (t + 2) % 4

            @pl.when(cur + 2 < ECH)
            def _(cur=cur, t2=t2):
                @pl.when(cur >= 2)
                def _(cur=cur, t2=t2):
                    wait_scat(t2)
                    issue_didx(cur + 2, t2)

                wait_idx(t2)
                issue_rows(t2, t2)

    for c in range(ECH - 4, ECH):
        wait_scat(c % 4)
    plsc.subcore_barrier()
    pltpu.sync_copy(acc.at[pl.ds(sid * ROWS_PER_TILE, ROWS_PER_TILE)],
                    out.at[cid, pl.ds(sid * ROWS_PER_TILE, ROWS_PER_TILE)])


def _spmm(pre, src_i, dst_i):
    return pl.kernel(
        _spmm_body,
        out_type=jax.ShapeDtypeStruct((NC, ACC_ROWS, D), jnp.float32),
        mesh=_mesh(),
        scratch_types=[
            pltpu.VMEM((4, EK), jnp.int32),
            pltpu.VMEM((4, EK), jnp.int32),
            pltpu.VMEM((EK, D), jnp.float32),
            pltpu.VMEM((EK, D), jnp.float32),
            pltpu.VMEM((EK, D), jnp.float32),
            pltpu.VMEM((EK, D), jnp.float32),
            pltpu.VMEM_SHARED((ACC_ROWS, D), jnp.float32),
        ] + [pltpu.SemaphoreType.DMA] * 16,
    )(pre, src_i, dst_i)

_BARRIER = "barrier-marker"
SEG_CH = (1, 2, 2, 20, 10)
GD_CH = 2


def _gather_body(h, gdir, sel2, col_i, neg_i, idn_i, gd_i,
                 o_sel, o_row, o_col, o_neg, o_idn, o_gd,
                 idx_v, rb0, rb1, rb2, rb3,
                 rsem0, rsem1, rsem2, rsem3, wsem0, wsem1, wsem2, wsem3):
    cid = lax.axis_index("c")
    sid = lax.axis_index("s")
    wid = cid * NS + sid
    rbufs = (rb0, rb1, rb2, rb3)
    rsems = (rsem0, rsem1, rsem2, rsem3)
    wsems = (wsem0, wsem1, wsem2, wsem3)

    for c2 in range(2):
        pltpu.sync_copy(sel2.at[sid * 2 + c2], idx_v.at[34 + c2])
        pltpu.async_copy(h.at[idx_v.at[34 + c2]], rbufs[c2], rsems[c2])
    for c2 in range(2):
        pltpu.make_async_copy(h.at[idx_v.at[34 + c2]], rbufs[c2],
                              rsems[c2]).wait()
        pltpu.sync_copy(rbufs[c2], o_sel.at[pl.ds((sid * 2 + c2) * K, K)])

    segs = ((col_i, o_col, SEG_CH[2], h),
            (neg_i, o_neg, SEG_CH[3], h),
            (idn_i, o_idn, SEG_CH[4], h),
            (gd_i, o_gd, GD_CH, gdir),
            (_BARRIER, o_row, SEG_CH[1], o_sel))

    off = 0
    for seg_i, seg_o, nch, table in segs:
        if seg_i is _BARRIER:
            plsc.subcore_barrier()
            seg_i = None
            off -= GD_CH
        if seg_i is not None:
            pltpu.sync_copy(seg_i.at[wid], idx_v.at[pl.ds(off, nch)])

        def issue(cur, b, table=table, off=off):
            pltpu.async_copy(table.at[idx_v.at[off + cur]], rbufs[b],
                             rsems[b])

        def wait_r(b, table=table, off=off):
            pltpu.make_async_copy(table.at[idx_v.at[off]], rbufs[b],
                                  rsems[b]).wait()

        def write(cur, b, seg_o=seg_o, nch=nch):
            pltpu.async_copy(rbufs[b],
                             seg_o.at[pl.ds(wid * nch * K + cur * K, K)],
                             wsems[b])

        def wait_w(b, seg_o=seg_o):
            pltpu.make_async_copy(rbufs[b], seg_o.at[pl.ds(0, K)],
                                  wsems[b]).wait()

        for c in range(min(2, nch)):
            issue(c, c)
        nfull = (nch // 4) * 4
        if nfull:
            @pl.loop(0, nfull, step=4)
            def _(j, issue=issue, wait_r=wait_r, write=write, wait_w=wait_w,
                  nch=nch):
                for t in range(4):
                    cur = j + t
                    wait_r(t)
                    write(cur, t)
                    t2 = (t + 2) % 4

                    @pl.when(cur + 2 < nch)
                    def _(cur=cur, t2=t2, issue=issue, wait_w=wait_w):
                        @pl.when(cur >= 2)
                        def _():
                            wait_w(t2)

                        issue(cur + 2, t2)

        for cur in range(nfull, nch):
            b = cur % 4
            wait_r(b)
            write(cur, b)
            if cur + 2 < nch:
                if cur >= 2:
                    wait_w((cur + 2) % 4)
                issue(cur + 2, (cur + 2) % 4)
        for c in range(max(0, nch - 4), nch):
            wait_w(c % 4)
        off += nch


_TOTAL_CH = 36

def _gather(h, gdir, sel2, col_i, neg_i, idn_i, gd_i):
    return pl.kernel(
        _gather_body,
        out_type=[
            jax.ShapeDtypeStruct((S, D), jnp.float32),
            jax.ShapeDtypeStruct((P, D), jnp.float32),
            jax.ShapeDtypeStruct((P, D), jnp.float32),
            jax.ShapeDtypeStruct((P * NEG, D), jnp.float32),
            jax.ShapeDtypeStruct((S * NEG, D), jnp.float32),
            jax.ShapeDtypeStruct((P, D), jnp.float32),
        ],
        mesh=_mesh(),
        scratch_types=[
            pltpu.VMEM((_TOTAL_CH, K), jnp.int32),
            pltpu.VMEM((K, D), jnp.float32),
            pltpu.VMEM((K, D), jnp.float32),
            pltpu.VMEM((K, D), jnp.float32),
            pltpu.VMEM((K, D), jnp.float32),
        ] + [pltpu.SemaphoreType.DMA] * 8,
    )(h, gdir, sel2, col_i, neg_i, idn_i, gd_i)


_MM_B = 1000


def _mm_body(x_ref, w_ref, o_ref):
    o_ref[...] = jnp.dot(x_ref[...], w_ref[...],
                         preferred_element_type=jnp.float32)


def _mm(x, w):
    return pl.pallas_call(
        _mm_body,
        grid=(N // _MM_B,),
        in_specs=[pl.BlockSpec((_MM_B, D), lambda i: (i, 0)),
                  pl.BlockSpec((D, D), lambda i: (0, 0))],
        out_specs=pl.BlockSpec((_MM_B, D), lambda i: (i, 0)),
        out_shape=jax.ShapeDtypeStruct((N, D), jnp.float32),
    )(x, w)


def _relu_mm_body(p_ref, w_ref, o_ref):
    s = jnp.maximum(p_ref[0] + p_ref[1], 0.0)
    o_ref[...] = jnp.dot(s, w_ref[...], preferred_element_type=jnp.float32)


def _relu_mm(parts, w):
    return pl.pallas_call(
        _relu_mm_body,
        grid=(N // _MM_B,),
        in_specs=[pl.BlockSpec((NC, _MM_B, D), lambda i: (0, i, 0)),
                  pl.BlockSpec((D, D), lambda i: (0, 0))],
        out_specs=pl.BlockSpec((_MM_B, D), lambda i: (i, 0)),
        out_shape=jax.ShapeDtypeStruct((N, D), jnp.float32),
    )(parts, w)


def _relu_add_body(p_ref, o_ref):
    o_ref[...] = jnp.maximum(p_ref[0] + p_ref[1], 0.0)


def _relu_add(parts):
    return pl.pallas_call(
        _relu_add_body,
        grid=(N // _MM_B,),
        in_specs=[pl.BlockSpec((NC, _MM_B, D), lambda i: (0, i, 0))],
        out_specs=pl.BlockSpec((_MM_B, D), lambda i: (i, 0)),
        out_shape=jax.ShapeDtypeStruct((N, D), jnp.float32),
    )(parts)


def _softplus(x):
    return jnp.maximum(x, 0.0) + jnp.log1p(jnp.exp(-jnp.abs(x)))


_LB = 1024


def _lane_consts():
    lane = lax.broadcasted_iota(jnp.int32, (1, 128), 1)
    w = jnp.where(lane == 0, 10.0, jnp.where(lane < 11, 1.0, 0.0))
    sgn = jnp.where(lane == 0, -1.0, jnp.where(lane < 11, 1.0, 0.0))
    return w, sgn


def _packed_logits(a, other, neg_ref, extra, bd):
    prods = [a * other] + [a * neg_ref[j] for j in range(NEG)] + [a * extra]
    x = jnp.concatenate(prods, axis=1)
    return jnp.dot(x, bd, preferred_element_type=jnp.float32)


def _loss_edges_body(row_ref, col_ref, neg_ref, gd_ref, w_ref, bd_ref,
                     og_ref, oa_ref):
    row = row_ref[...]
    col = col_ref[...]
    gdir = gd_ref[...]
    wstd = jnp.broadcast_to(w_ref[...], row.shape)
    bd = bd_ref[...]
    w, sgn = _lane_consts()

    p1 = _packed_logits(row, col, neg_ref, wstd, bd)
    p2 = _packed_logits(gdir, col, neg_ref, gdir, bd)
    std = jax.nn.sigmoid(p1[:, 11:12])
    c = std / jnp.maximum(jnp.sqrt(p2[:, 11:12]), 1e-12)
    base = p1 * sgn
    aug = (p1 + c * p2) * sgn
    gae_sum = jnp.sum(w * _softplus(base))
    aug_sum = jnp.sum(w * _softplus(aug))

    @pl.when(pl.program_id(0) == 0)
    def _():
        og_ref[...] = jnp.zeros_like(og_ref)
        oa_ref[...] = jnp.zeros_like(oa_ref)

    og_ref[...] = og_ref[...] + gae_sum
    oa_ref[...] = oa_ref[...] + aug_sum


def _loss_edges(row_h, col_h, neg3, gdir_rows, wstd_t, bd):
    return pl.pallas_call(
        _loss_edges_body,
        grid=(P // _LB,),
        in_specs=[pl.BlockSpec((_LB, D), lambda i: (i, 0)),
                  pl.BlockSpec((_LB, D), lambda i: (i, 0)),
                  pl.BlockSpec((NEG, _LB, D), lambda i: (0, i, 0)),
                  pl.BlockSpec((_LB, D), lambda i: (i, 0)),
                  pl.BlockSpec((1, D), lambda i: (0, 0)),
                  pl.BlockSpec((12 * D, 128), lambda i: (0, 0))],
        out_specs=[pl.BlockSpec((1, 1), lambda i: (0, 0)),
                   pl.BlockSpec((1, 1), lambda i: (0, 0))],
        out_shape=[jax.ShapeDtypeStruct((1, 1), jnp.float32),
                   jax.ShapeDtypeStruct((1, 1), jnp.float32)],
    )(row_h, col_h, neg3, gdir_rows, wstd_t, bd)


def _loss_inst_body(gae_ref, gdir_ref, neg_ref, w_ref, bd_ref, o_ref):
    gae_h = gae_ref[...]
    gdir = gdir_ref[...]
    wstd = jnp.broadcast_to(w_ref[...], gae_h.shape)
    bd = bd_ref[...]
    w, sgn = _lane_consts()
    w = jnp.where(w > 0.0, 1.0, 0.0)

    p1 = _packed_logits(gae_h, gae_h, neg_ref, wstd, bd)
    p2 = _packed_logits(gdir, gae_h, neg_ref, gdir, bd)
    std = jax.nn.sigmoid(p1[:, 11:12])
    c = std / jnp.maximum(jnp.sqrt(p2[:, 11:12]), 1e-12)
    logits = ((p1 + c * p2) * (1.0 / TEMP)) * sgn
    acc = jnp.sum(w * _softplus(logits))

    @pl.when(pl.program_id(0) == 0)
    def _():
        o_ref[...] = jnp.zeros_like(o_ref)

    o_ref[...] = o_ref[...] + acc


def _loss_inst(gae_h, gdir, idn3, wstd_t, bd):
    return pl.pallas_call(
        _loss_inst_body,
        grid=(S // _LB,),
        in_specs=[pl.BlockSpec((_LB, D), lambda i: (i, 0)),
                  pl.BlockSpec((_LB, D), lambda i: (i, 0)),
                  pl.BlockSpec((NEG, _LB, D), lambda i: (0, i, 0)),
                  pl.BlockSpec((1, D), lambda i: (0, 0)),
                  pl.BlockSpec((12 * D, 128), lambda i: (0, 0))],
        out_specs=pl.BlockSpec((1, 1), lambda i: (0, 0)),
        out_shape=jax.ShapeDtypeStruct((1, 1), jnp.float32),
    )(gae_h, gdir, idn3, wstd_t, bd)


def kernel(x, edge_index, gradint_dir, select_index, id2id, new_rows, cols,
           negs, id_negs, W1, W2, W_std):
    src = edge_index[0]
    dst = edge_index[1]
    pad = PAD_E - NE
    pad_i = jnp.arange(pad, dtype=jnp.int32)
    src_p = jnp.concatenate([src, pad_i % N]).reshape(NW, ECH, EK)
    dst_p = jnp.concatenate([dst, TRASH + pad_i % (ACC_ROWS - N)]).reshape(NW, ECH, EK)

    pre1 = _mm(x, W1)
    parts1 = _spmm(pre1, src_p, dst_p)
    pre2 = _relu_mm(parts1, W2)
    parts2 = _spmm(pre2, src_p, dst_p)
    h = _relu_add(parts2)

    sel_i = select_index.reshape(NW, K)
    col_i = cols.reshape(NW, SEG_CH[2], K)
    neg_i = negs.T.reshape(NW, SEG_CH[3], K)
    idn_i = id_negs.T.reshape(NW, SEG_CH[4], K)
    gd_i = new_rows.reshape(NW, GD_CH, K)

    gae_h, row_h, col_h, neg_f, idn_f, gdir_rows = _gather(
        h, gradint_dir, sel_i, col_i, neg_i, idn_i, gd_i)

    neg3 = neg_f.reshape(NEG, P, D)
    idn3 = idn_f.reshape(NEG, S, D)
    wstd_t = W_std.reshape(1, D)
    bd = jnp.repeat(jnp.eye(12, 128, dtype=jnp.float32), D, axis=0)

    lg, la = _loss_edges(row_h, col_h, neg3, gdir_rows, wstd_t, bd)
    l2 = _loss_inst(gae_h, gradint_dir, idn3, wstd_t, bd)

    gae_loss = lg[0, 0] / P
    aug_loss = la[0, 0] / P
    inst_loss = l2[0, 0] / S
    return gae_loss + AUG_W * aug_loss + INS_W * inst_loss


# speedup vs baseline: 10.6183x; 1.0179x over previous
import functools

import jax
import jax.numpy as jnp
from jax import lax
from jax.experimental import pallas as pl
from jax.experimental.pallas import tpu as pltpu
from jax.experimental.pallas import tpu_sc as plsc

N = 10000
D = 128
NE = 320000
S = 4096
P = 8192
NEG = 10
TEMP = 0.07
AUG_W = 1e-05
INS_W = 1e-05

NC, NS, L = 2, 16, 16
NW = NC * NS
K = 128
EK = 64
ECH = 160
EW = ECH * EK
PAD_E = NW * EW
ACC_ROWS = 10240
ROWS_PER_TILE = ACC_ROWS // NS
TRASH = N

def _mesh():
    return plsc.VectorSubcoreMesh(
        core_axis_name="c", subcore_axis_name="s", num_cores=NC, num_subcores=NS)


def _spmm_body(pre, src_i, dst_i, out, sbuf, dbuf, rb0, rb1, rb2, rb3, acc,
               rsem0, rsem1, rsem2, rsem3, isem0, isem1, isem2, isem3,
               ssem0, ssem1, ssem2, ssem3, dsem0, dsem1, dsem2, dsem3):
    cid = lax.axis_index("c")
    sid = lax.axis_index("s")
    wid = cid * NS + sid
    rbufs = (rb0, rb1, rb2, rb3)
    rsems = (rsem0, rsem1, rsem2, rsem3)
    isems = (isem0, isem1, isem2, isem3)
    ssems = (ssem0, ssem1, ssem2, ssem3)
    dsems = (dsem0, dsem1, dsem2, dsem3)

    def issue_idx(cur, s):
        pltpu.async_copy(src_i.at[wid, cur], sbuf.at[s], isems[s])

    def wait_idx(s):
        pltpu.make_async_copy(src_i.at[wid, 0], sbuf.at[s], isems[s]).wait()

    def issue_didx(cur, s):
        pltpu.async_copy(dst_i.at[wid, cur], dbuf.at[s], dsems[s])

    def wait_didx(s):
        pltpu.make_async_copy(dst_i.at[wid, 0], dbuf.at[s], dsems[s]).wait()

    def issue_rows(s, b):
        pltpu.async_copy(pre.at[sbuf.at[s]], rbufs[b], rsems[b])

    def wait_rows(b):
        pltpu.make_async_copy(pre.at[sbuf.at[b]], rbufs[b], rsems[b]).wait()

    def issue_scat(b):
        pltpu.async_copy(rbufs[b], acc.at[dbuf.at[b]], ssems[b], add=True)

    def wait_scat(b):
        pltpu.make_async_copy(rbufs[b], acc.at[dbuf.at[b]], ssems[b]).wait()

    for c in range(4):
        issue_idx(c, c)
        issue_didx(c, c)
    for c in range(2):
        wait_idx(c)
        issue_rows(c, c)

    @pl.loop(0, EK)
    def _(r):
        for c in range(D // L):
            rb3[r, pl.ds(c * L, L)] = jnp.zeros((L,), jnp.float32)

    for c in range(ROWS_PER_TILE // EK):
        pltpu.sync_copy(rb3, acc.at[pl.ds(sid * ROWS_PER_TILE + c * EK, EK)])
    plsc.subcore_barrier()

    @pl.loop(0, ECH, step=4)
    def _(j):
        for t in range(4):
            cur = j + t
            wait_rows(t)

            @pl.when(cur + 4 < ECH)
            def _(cur=cur, t=t):
                issue_idx(cur + 4, t)

            wait_didx(t)
            issue_scat(t)
            t2 = (t + 2) % 4

            @pl.when(cur + 2 < ECH)
            def _(cur=cur, t2=t2):
                @pl.when(cur >= 2)
                def _(cur=cur, t2=t2):
                    wait_scat(t2)
                    issue_didx(cur + 2, t2)

                wait_idx(t2)
                issue_rows(t2, t2)

    for c in range(ECH - 4, ECH):
        wait_scat(c % 4)
    plsc.subcore_barrier()
    pltpu.sync_copy(acc.at[pl.ds(sid * ROWS_PER_TILE, ROWS_PER_TILE)],
                    out.at[cid, pl.ds(sid * ROWS_PER_TILE, ROWS_PER_TILE)])


def _spmm(pre, src_i, dst_i):
    return pl.kernel(
        _spmm_body,
        out_type=jax.ShapeDtypeStruct((NC, ACC_ROWS, D), jnp.float32),
        mesh=_mesh(),
        scratch_types=[
            pltpu.VMEM((4, EK), jnp.int32),
            pltpu.VMEM((4, EK), jnp.int32),
            pltpu.VMEM((EK, D), jnp.float32),
            pltpu.VMEM((EK, D), jnp.float32),
            pltpu.VMEM((EK, D), jnp.float32),
            pltpu.VMEM((EK, D), jnp.float32),
            pltpu.VMEM_SHARED((ACC_ROWS, D), jnp.float32),
        ] + [pltpu.SemaphoreType.DMA] * 16,
    )(pre, src_i, dst_i)

SEG_CH = (1, 2, 2, 20, 10)
GD_CH = 2


def _seg_ring(table, idx_v, off, nch, seg_o, wid, rbufs, rsems, wsems):
    def issue(cur, b):
        pltpu.async_copy(table.at[idx_v.at[off + cur]], rbufs[b], rsems[b])

    def wait_r(b):
        pltpu.make_async_copy(table.at[idx_v.at[off]], rbufs[b],
                              rsems[b]).wait()

    def write(cur, b):
        pltpu.async_copy(rbufs[b],
                         seg_o.at[pl.ds(wid * nch * K + cur * K, K)],
                         wsems[b])

    def wait_w(b):
        pltpu.make_async_copy(rbufs[b], seg_o.at[pl.ds(0, K)],
                              wsems[b]).wait()

    for c in range(min(2, nch)):
        issue(c, c)
    nfull = (nch // 4) * 4
    if nfull:
        @pl.loop(0, nfull, step=4)
        def _(j):
            for t in range(4):
                cur = j + t
                wait_r(t)
                write(cur, t)
                t2 = (t + 2) % 4

                @pl.when(cur + 2 < nch)
                def _(cur=cur, t2=t2):
                    @pl.when(cur >= 2)
                    def _():
                        wait_w(t2)

                    issue(cur + 2, t2)

    for cur in range(nfull, nch):
        b = cur % 4
        wait_r(b)
        write(cur, b)
        if cur + 2 < nch:
            if cur >= 2:
                wait_w((cur + 2) % 4)
            issue(cur + 2, (cur + 2) % 4)
    for c in range(max(0, nch - 4), nch):
        wait_w(c % 4)


def _gather1_body(h, sel_i, idn_i, o_sel, o_idn,
                  idx_v, rb0, rb1, rb2, rb3,
                  rsem0, rsem1, rsem2, rsem3, wsem0, wsem1, wsem2, wsem3):
    cid = lax.axis_index("c")
    sid = lax.axis_index("s")
    wid = cid * NS + sid
    rbufs = (rb0, rb1, rb2, rb3)
    rsems = (rsem0, rsem1, rsem2, rsem3)
    wsems = (wsem0, wsem1, wsem2, wsem3)

    pltpu.sync_copy(sel_i.at[wid], idx_v.at[pl.ds(0, 1)])
    pltpu.sync_copy(idn_i.at[wid], idx_v.at[pl.ds(1, SEG_CH[4])])
    _seg_ring(h, idx_v, 0, 1, o_sel, wid, rbufs, rsems, wsems)
    _seg_ring(h, idx_v, 1, SEG_CH[4], o_idn, wid, rbufs, rsems, wsems)


def _gather1(h, sel_i, idn_i):
    return pl.kernel(
        _gather1_body,
        out_type=[
            jax.ShapeDtypeStruct((S, D), jnp.float32),
            jax.ShapeDtypeStruct((S * NEG, D), jnp.float32),
        ],
        mesh=_mesh(),
        scratch_types=[
            pltpu.VMEM((1 + SEG_CH[4], K), jnp.int32),
            pltpu.VMEM((K, D), jnp.float32),
            pltpu.VMEM((K, D), jnp.float32),
            pltpu.VMEM((K, D), jnp.float32),
            pltpu.VMEM((K, D), jnp.float32),
        ] + [pltpu.SemaphoreType.DMA] * 8,
    )(h, sel_i, idn_i)


def _gather2_body(h, gdir, osel, col_i, neg_i, gd_i,
                  o_row, o_col, o_neg, o_gd,
                  idx_v, rb0, rb1, rb2, rb3,
                  rsem0, rsem1, rsem2, rsem3, wsem0, wsem1, wsem2, wsem3):
    cid = lax.axis_index("c")
    sid = lax.axis_index("s")
    wid = cid * NS + sid
    rbufs = (rb0, rb1, rb2, rb3)
    rsems = (rsem0, rsem1, rsem2, rsem3)
    wsems = (wsem0, wsem1, wsem2, wsem3)

    pltpu.sync_copy(col_i.at[wid], idx_v.at[pl.ds(0, SEG_CH[2])])
    pltpu.sync_copy(neg_i.at[wid], idx_v.at[pl.ds(2, SEG_CH[3])])
    pltpu.sync_copy(gd_i.at[wid], idx_v.at[pl.ds(22, GD_CH)])
    _seg_ring(h, idx_v, 0, SEG_CH[2], o_col, wid, rbufs, rsems, wsems)
    _seg_ring(h, idx_v, 2, SEG_CH[3], o_neg, wid, rbufs, rsems, wsems)
    _seg_ring(gdir, idx_v, 22, GD_CH, o_gd, wid, rbufs, rsems, wsems)
    _seg_ring(osel, idx_v, 22, SEG_CH[1], o_row, wid, rbufs, rsems, wsems)


def _gather2(h, gdir, osel, col_i, neg_i, gd_i):
    return pl.kernel(
        _gather2_body,
        out_type=[
            jax.ShapeDtypeStruct((P, D), jnp.float32),
            jax.ShapeDtypeStruct((P, D), jnp.float32),
            jax.ShapeDtypeStruct((P * NEG, D), jnp.float32),
            jax.ShapeDtypeStruct((P, D), jnp.float32),
        ],
        mesh=_mesh(),
        scratch_types=[
            pltpu.VMEM((22 + GD_CH, K), jnp.int32),
            pltpu.VMEM((K, D), jnp.float32),
            pltpu.VMEM((K, D), jnp.float32),
            pltpu.VMEM((K, D), jnp.float32),
            pltpu.VMEM((K, D), jnp.float32),
        ] + [pltpu.SemaphoreType.DMA] * 8,
    )(h, gdir, osel, col_i, neg_i, gd_i)


_MM_B = 1000


def _mm_body(x_ref, w_ref, o_ref):
    o_ref[...] = jnp.dot(x_ref[...], w_ref[...],
                         preferred_element_type=jnp.float32)


def _mm(x, w):
    return pl.pallas_call(
        _mm_body,
        grid=(N // _MM_B,),
        in_specs=[pl.BlockSpec((_MM_B, D), lambda i: (i, 0)),
                  pl.BlockSpec((D, D), lambda i: (0, 0))],
        out_specs=pl.BlockSpec((_MM_B, D), lambda i: (i, 0)),
        out_shape=jax.ShapeDtypeStruct((N, D), jnp.float32),
    )(x, w)


def _relu_mm_body(p_ref, w_ref, o_ref):
    s = jnp.maximum(p_ref[0] + p_ref[1], 0.0)
    o_ref[...] = jnp.dot(s, w_ref[...], preferred_element_type=jnp.float32)


def _relu_mm(parts, w):
    return pl.pallas_call(
        _relu_mm_body,
        grid=(N // _MM_B,),
        in_specs=[pl.BlockSpec((NC, _MM_B, D), lambda i: (0, i, 0)),
                  pl.BlockSpec((D, D), lambda i: (0, 0))],
        out_specs=pl.BlockSpec((_MM_B, D), lambda i: (i, 0)),
        out_shape=jax.ShapeDtypeStruct((N, D), jnp.float32),
    )(parts, w)


def _relu_add_body(p_ref, o_ref):
    o_ref[...] = jnp.maximum(p_ref[0] + p_ref[1], 0.0)


def _relu_add(parts):
    return pl.pallas_call(
        _relu_add_body,
        grid=(N // _MM_B,),
        in_specs=[pl.BlockSpec((NC, _MM_B, D), lambda i: (0, i, 0))],
        out_specs=pl.BlockSpec((_MM_B, D), lambda i: (i, 0)),
        out_shape=jax.ShapeDtypeStruct((N, D), jnp.float32),
    )(parts)


def _softplus(x):
    return jnp.maximum(x, 0.0) + jnp.log1p(jnp.exp(-jnp.abs(x)))


_LB = 1024


def _lane_consts():
    lane = lax.broadcasted_iota(jnp.int32, (1, 128), 1)
    w = jnp.where(lane == 0, 10.0, jnp.where(lane < 11, 1.0, 0.0))
    sgn = jnp.where(lane == 0, -1.0, jnp.where(lane < 11, 1.0, 0.0))
    return w, sgn


def _packed_logits(a, other, neg_ref, extra, bd):
    prods = [a * other] + [a * neg_ref[j] for j in range(NEG)] + [a * extra]
    x = jnp.concatenate(prods, axis=1)
    return jnp.dot(x, bd, preferred_element_type=jnp.float32)


def _loss_edges_body(row_ref, col_ref, neg_ref, gd_ref, w_ref, bd_ref,
                     og_ref, oa_ref):
    row = row_ref[...]
    col = col_ref[...]
    gdir = gd_ref[...]
    wstd = jnp.broadcast_to(w_ref[...], row.shape)
    bd = bd_ref[...]
    w, sgn = _lane_consts()

    p1 = _packed_logits(row, col, neg_ref, wstd, bd)
    p2 = _packed_logits(gdir, col, neg_ref, gdir, bd)
    std = jax.nn.sigmoid(p1[:, 11:12])
    c = std / jnp.maximum(jnp.sqrt(p2[:, 11:12]), 1e-12)
    base = p1 * sgn
    aug = (p1 + c * p2) * sgn
    gae_sum = jnp.sum(w * _softplus(base))
    aug_sum = jnp.sum(w * _softplus(aug))

    @pl.when(pl.program_id(0) == 0)
    def _():
        og_ref[...] = jnp.zeros_like(og_ref)
        oa_ref[...] = jnp.zeros_like(oa_ref)

    og_ref[...] = og_ref[...] + gae_sum
    oa_ref[...] = oa_ref[...] + aug_sum


def _loss_edges(row_h, col_h, neg3, gdir_rows, wstd_t, bd):
    return pl.pallas_call(
        _loss_edges_body,
        grid=(P // _LB,),
        in_specs=[pl.BlockSpec((_LB, D), lambda i: (i, 0)),
                  pl.BlockSpec((_LB, D), lambda i: (i, 0)),
                  pl.BlockSpec((NEG, _LB, D), lambda i: (0, i, 0)),
                  pl.BlockSpec((_LB, D), lambda i: (i, 0)),
                  pl.BlockSpec((1, D), lambda i: (0, 0)),
                  pl.BlockSpec((12 * D, 128), lambda i: (0, 0))],
        out_specs=[pl.BlockSpec((1, 1), lambda i: (0, 0)),
                   pl.BlockSpec((1, 1), lambda i: (0, 0))],
        out_shape=[jax.ShapeDtypeStruct((1, 1), jnp.float32),
                   jax.ShapeDtypeStruct((1, 1), jnp.float32)],
    )(row_h, col_h, neg3, gdir_rows, wstd_t, bd)


def _loss_inst_body(gae_ref, gdir_ref, neg_ref, w_ref, bd_ref, o_ref):
    gae_h = gae_ref[...]
    gdir = gdir_ref[...]
    wstd = jnp.broadcast_to(w_ref[...], gae_h.shape)
    bd = bd_ref[...]
    w, sgn = _lane_consts()
    w = jnp.where(w > 0.0, 1.0, 0.0)

    p1 = _packed_logits(gae_h, gae_h, neg_ref, wstd, bd)
    p2 = _packed_logits(gdir, gae_h, neg_ref, gdir, bd)
    std = jax.nn.sigmoid(p1[:, 11:12])
    c = std / jnp.maximum(jnp.sqrt(p2[:, 11:12]), 1e-12)
    logits = ((p1 + c * p2) * (1.0 / TEMP)) * sgn
    acc = jnp.sum(w * _softplus(logits))

    @pl.when(pl.program_id(0) == 0)
    def _():
        o_ref[...] = jnp.zeros_like(o_ref)

    o_ref[...] = o_ref[...] + acc


def _loss_inst(gae_h, gdir, idn3, wstd_t, bd):
    return pl.pallas_call(
        _loss_inst_body,
        grid=(S // _LB,),
        in_specs=[pl.BlockSpec((_LB, D), lambda i: (i, 0)),
                  pl.BlockSpec((_LB, D), lambda i: (i, 0)),
                  pl.BlockSpec((NEG, _LB, D), lambda i: (0, i, 0)),
                  pl.BlockSpec((1, D), lambda i: (0, 0)),
                  pl.BlockSpec((12 * D, 128), lambda i: (0, 0))],
        out_specs=pl.BlockSpec((1, 1), lambda i: (0, 0)),
        out_shape=jax.ShapeDtypeStruct((1, 1), jnp.float32),
    )(gae_h, gdir, idn3, wstd_t, bd)


def kernel(x, edge_index, gradint_dir, select_index, id2id, new_rows, cols,
           negs, id_negs, W1, W2, W_std):
    src = edge_index[0]
    dst = edge_index[1]
    pad = PAD_E - NE
    pad_i = jnp.arange(pad, dtype=jnp.int32)
    src_p = jnp.concatenate([src, pad_i % N]).reshape(NW, ECH, EK)
    dst_p = jnp.concatenate([dst, TRASH + pad_i % (ACC_ROWS - N)]).reshape(NW, ECH, EK)

    pre1 = _mm(x, W1)
    parts1 = _spmm(pre1, src_p, dst_p)
    pre2 = _relu_mm(parts1, W2)
    parts2 = _spmm(pre2, src_p, dst_p)
    h = _relu_add(parts2)

    sel_i = select_index.reshape(NW, 1, K)
    col_i = cols.reshape(NW, SEG_CH[2], K)
    neg_i = negs.T.reshape(NW, SEG_CH[3], K)
    idn_i = id_negs.T.reshape(NW, SEG_CH[4], K)
    gd_i = new_rows.reshape(NW, GD_CH, K)

    wstd_t = W_std.reshape(1, D)
    bd = jnp.repeat(jnp.eye(12, 128, dtype=jnp.float32), D, axis=0)

    gae_h, idn_f = _gather1(h, sel_i, idn_i)
    idn3 = idn_f.reshape(NEG, S, D)
    l2 = _loss_inst(gae_h, gradint_dir, idn3, wstd_t, bd)

    row_h, col_h, neg_f, gdir_rows = _gather2(
        h, gradint_dir, gae_h, col_i, neg_i, gd_i)
    neg3 = neg_f.reshape(NEG, P, D)
    lg, la = _loss_edges(row_h, col_h, neg3, gdir_rows, wstd_t, bd)

    gae_loss = lg[0, 0] / P
    aug_loss = la[0, 0] / P
    inst_loss = l2[0, 0] / S
    return gae_loss + AUG_W * aug_loss + INS_W * inst_loss


# speedup vs baseline: 10.8252x; 1.0195x over previous
import functools

import jax
import jax.numpy as jnp
from jax import lax
from jax.experimental import pallas as pl
from jax.experimental.pallas import tpu as pltpu
from jax.experimental.pallas import tpu_sc as plsc

N = 10000
D = 128
NE = 320000
S = 4096
P = 8192
NEG = 10
TEMP = 0.07
AUG_W = 1e-05
INS_W = 1e-05

NC, NS, L = 2, 16, 16
NW = NC * NS
K = 128
EK = 64
ECH = 160
EW = ECH * EK
PAD_E = NW * EW
ACC_ROWS = 10240
ROWS_PER_TILE = ACC_ROWS // NS
TRASH = N

def _mesh():
    return plsc.VectorSubcoreMesh(
        core_axis_name="c", subcore_axis_name="s", num_cores=NC, num_subcores=NS)


def _spmm_body(pre, src_i, dst_i, out, sbuf, dbuf, rb0, rb1, rb2, rb3, acc,
               rsem0, rsem1, rsem2, rsem3, isem0, isem1, isem2, isem3,
               ssem0, ssem1, ssem2, ssem3, dsem0, dsem1, dsem2, dsem3):
    cid = lax.axis_index("c")
    sid = lax.axis_index("s")
    wid = cid * NS + sid
    rbufs = (rb0, rb1, rb2, rb3)
    rsems = (rsem0, rsem1, rsem2, rsem3)
    isems = (isem0, isem1, isem2, isem3)
    ssems = (ssem0, ssem1, ssem2, ssem3)
    dsems = (dsem0, dsem1, dsem2, dsem3)

    def issue_idx(cur, s):
        pltpu.async_copy(src_i.at[wid, cur], sbuf.at[s], isems[s])

    def wait_idx(s):
        pltpu.make_async_copy(src_i.at[wid, 0], sbuf.at[s], isems[s]).wait()

    def issue_didx(cur, s):
        pltpu.async_copy(dst_i.at[wid, cur], dbuf.at[s], dsems[s])

    def wait_didx(s):
        pltpu.make_async_copy(dst_i.at[wid, 0], dbuf.at[s], dsems[s]).wait()

    def issue_rows(s, b):
        pltpu.async_copy(pre.at[sbuf.at[s]], rbufs[b], rsems[b])

    def wait_rows(b):
        pltpu.make_async_copy(pre.at[sbuf.at[b]], rbufs[b], rsems[b]).wait()

    def issue_scat(b):
        pltpu.async_copy(rbufs[b], acc.at[dbuf.at[b]], ssems[b], add=True)

    def wait_scat(b):
        pltpu.make_async_copy(rbufs[b], acc.at[dbuf.at[b]], ssems[b]).wait()

    for c in range(4):
        issue_idx(c, c)
        issue_didx(c, c)
    for c in range(2):
        wait_idx(c)
        issue_rows(c, c)

    @pl.loop(0, EK)
    def _(r):
        for c in range(D // L):
            rb3[r, pl.ds(c * L, L)] = jnp.zeros((L,), jnp.float32)

    for c in range(ROWS_PER_TILE // EK):
        pltpu.sync_copy(rb3, acc.at[pl.ds(sid * ROWS_PER_TILE + c * EK, EK)])
    plsc.subcore_barrier()

    @pl.loop(0, ECH, step=4)
    def _(j):
        for t in range(4):
            cur = j + t
            wait_rows(t)

            @pl.when(cur + 4 < ECH)
            def _(cur=cur, t=t):
                issue_idx(cur + 4, t)

            wait_didx(t)
            issue_scat(t)
            t2 = (t + 2) % 4

            @pl.when(cur + 2 < ECH)
            def _(cur=cur, t2=t2):
                @pl.when(cur >= 2)
                def _(cur=cur, t2=t2):
                    wait_scat(t2)
                    issue_didx(cur + 2, t2)

                wait_idx(t2)
                issue_rows(t2, t2)

    for c in range(ECH - 4, ECH):
        wait_scat(c % 4)
    plsc.subcore_barrier()
    pltpu.sync_copy(acc.at[pl.ds(sid * ROWS_PER_TILE, ROWS_PER_TILE)],
                    out.at[cid, pl.ds(sid * ROWS_PER_TILE, ROWS_PER_TILE)])


def _spmm(pre, src_i, dst_i):
    return pl.kernel(
        _spmm_body,
        out_type=jax.ShapeDtypeStruct((NC, ACC_ROWS, D), jnp.float32),
        mesh=_mesh(),
        scratch_types=[
            pltpu.VMEM((4, EK), jnp.int32),
            pltpu.VMEM((4, EK), jnp.int32),
            pltpu.VMEM((EK, D), jnp.float32),
            pltpu.VMEM((EK, D), jnp.float32),
            pltpu.VMEM((EK, D), jnp.float32),
            pltpu.VMEM((EK, D), jnp.float32),
            pltpu.VMEM_SHARED((ACC_ROWS, D), jnp.float32),
        ] + [pltpu.SemaphoreType.DMA] * 16,
    )(pre, src_i, dst_i)

SEG_CH = (1, 2, 2, 20, 10)
GD_CH = 2


def _seg_ring(table, idx_v, off, nch, seg_o, wid, rbufs, rsems, wsems):
    def issue(cur, b):
        pltpu.async_copy(table.at[idx_v.at[off + cur]], rbufs[b], rsems[b])

    def wait_r(b):
        pltpu.make_async_copy(table.at[idx_v.at[off]], rbufs[b],
                              rsems[b]).wait()

    def write(cur, b):
        pltpu.async_copy(rbufs[b],
                         seg_o.at[pl.ds(wid * nch * K + cur * K, K)],
                         wsems[b])

    def wait_w(b):
        pltpu.make_async_copy(rbufs[b], seg_o.at[pl.ds(0, K)],
                              wsems[b]).wait()

    for c in range(min(2, nch)):
        issue(c, c)
    nfull = (nch // 4) * 4
    if nfull:
        @pl.loop(0, nfull, step=4)
        def _(j):
            for t in range(4):
                cur = j + t
                wait_r(t)
                write(cur, t)
                t2 = (t + 2) % 4

                @pl.when(cur + 2 < nch)
                def _(cur=cur, t2=t2):
                    @pl.when(cur >= 2)
                    def _():
                        wait_w(t2)

                    issue(cur + 2, t2)

    for cur in range(nfull, nch):
        b = cur % 4
        wait_r(b)
        write(cur, b)
        if cur + 2 < nch:
            if cur >= 2:
                wait_w((cur + 2) % 4)
            issue(cur + 2, (cur + 2) % 4)
    for c in range(max(0, nch - 4), nch):
        wait_w(c % 4)


def _gather1_body(h, sel_i, idn_i, o_sel, o_idn,
                  idx_v, rb0, rb1, rb2, rb3,
                  rsem0, rsem1, rsem2, rsem3, wsem0, wsem1, wsem2, wsem3):
    cid = lax.axis_index("c")
    sid = lax.axis_index("s")
    wid = cid * NS + sid
    rbufs = (rb0, rb1, rb2, rb3)
    rsems = (rsem0, rsem1, rsem2, rsem3)
    wsems = (wsem0, wsem1, wsem2, wsem3)

    pltpu.sync_copy(sel_i.at[wid], idx_v.at[pl.ds(0, 1)])
    pltpu.sync_copy(idn_i.at[wid], idx_v.at[pl.ds(1, SEG_CH[4])])
    _seg_ring(h, idx_v, 0, 1, o_sel, wid, rbufs, rsems, wsems)
    _seg_ring(h, idx_v, 1, SEG_CH[4], o_idn, wid, rbufs, rsems, wsems)


def _gather1(h, sel_i, idn_i):
    return pl.kernel(
        _gather1_body,
        out_type=[
            jax.ShapeDtypeStruct((S, D), jnp.float32),
            jax.ShapeDtypeStruct((S * NEG, D), jnp.float32),
        ],
        mesh=_mesh(),
        scratch_types=[
            pltpu.VMEM((1 + SEG_CH[4], K), jnp.int32),
            pltpu.VMEM((K, D), jnp.float32),
            pltpu.VMEM((K, D), jnp.float32),
            pltpu.VMEM((K, D), jnp.float32),
            pltpu.VMEM((K, D), jnp.float32),
        ] + [pltpu.SemaphoreType.DMA] * 8,
    )(h, sel_i, idn_i)


def _gather2_body(h, gdir, osel, col_i, neg_i, gd_i,
                  o_row, o_col, o_neg, o_gd,
                  idx_v, rb0, rb1, rb2, rb3,
                  rsem0, rsem1, rsem2, rsem3, wsem0, wsem1, wsem2, wsem3):
    cid = lax.axis_index("c")
    sid = lax.axis_index("s")
    wid = cid * NS + sid
    rbufs = (rb0, rb1, rb2, rb3)
    rsems = (rsem0, rsem1, rsem2, rsem3)
    wsems = (wsem0, wsem1, wsem2, wsem3)

    pltpu.sync_copy(col_i.at[wid], idx_v.at[pl.ds(0, SEG_CH[2])])
    pltpu.sync_copy(neg_i.at[wid], idx_v.at[pl.ds(2, SEG_CH[3])])
    pltpu.sync_copy(gd_i.at[wid], idx_v.at[pl.ds(22, GD_CH)])
    _seg_ring(h, idx_v, 0, SEG_CH[2], o_col, wid, rbufs, rsems, wsems)
    _seg_ring(h, idx_v, 2, SEG_CH[3], o_neg, wid, rbufs, rsems, wsems)
    _seg_ring(gdir, idx_v, 22, GD_CH, o_gd, wid, rbufs, rsems, wsems)
    _seg_ring(osel, idx_v, 22, SEG_CH[1], o_row, wid, rbufs, rsems, wsems)


def _gather2(h, gdir, osel, col_i, neg_i, gd_i):
    return pl.kernel(
        _gather2_body,
        out_type=[
            jax.ShapeDtypeStruct((P, D), jnp.float32),
            jax.ShapeDtypeStruct((P, D), jnp.float32),
            jax.ShapeDtypeStruct((P * NEG, D), jnp.float32),
            jax.ShapeDtypeStruct((P, D), jnp.float32),
        ],
        mesh=_mesh(),
        scratch_types=[
            pltpu.VMEM((22 + GD_CH, K), jnp.int32),
            pltpu.VMEM((K, D), jnp.float32),
            pltpu.VMEM((K, D), jnp.float32),
            pltpu.VMEM((K, D), jnp.float32),
            pltpu.VMEM((K, D), jnp.float32),
        ] + [pltpu.SemaphoreType.DMA] * 8,
    )(h, gdir, osel, col_i, neg_i, gd_i)


_MM_B = 2000


def _mm_body(x_ref, w_ref, o_ref):
    o_ref[...] = jnp.dot(x_ref[...], w_ref[...],
                         preferred_element_type=jnp.float32)


def _mm(x, w):
    return pl.pallas_call(
        _mm_body,
        grid=(N // _MM_B,),
        in_specs=[pl.BlockSpec((_MM_B, D), lambda i: (i, 0)),
                  pl.BlockSpec((D, D), lambda i: (0, 0))],
        out_specs=pl.BlockSpec((_MM_B, D), lambda i: (i, 0)),
        out_shape=jax.ShapeDtypeStruct((N, D), jnp.float32),
    )(x, w)


def _relu_mm_body(p_ref, w_ref, o_ref):
    s = jnp.maximum(p_ref[0] + p_ref[1], 0.0)
    o_ref[...] = jnp.dot(s, w_ref[...], preferred_element_type=jnp.float32)


def _relu_mm(parts, w):
    return pl.pallas_call(
        _relu_mm_body,
        grid=(N // _MM_B,),
        in_specs=[pl.BlockSpec((NC, _MM_B, D), lambda i: (0, i, 0)),
                  pl.BlockSpec((D, D), lambda i: (0, 0))],
        out_specs=pl.BlockSpec((_MM_B, D), lambda i: (i, 0)),
        out_shape=jax.ShapeDtypeStruct((N, D), jnp.float32),
    )(parts, w)


def _relu_add_body(p_ref, o_ref):
    o_ref[...] = jnp.maximum(p_ref[0] + p_ref[1], 0.0)


def _relu_add(parts):
    return pl.pallas_call(
        _relu_add_body,
        grid=(N // _MM_B,),
        in_specs=[pl.BlockSpec((NC, _MM_B, D), lambda i: (0, i, 0))],
        out_specs=pl.BlockSpec((_MM_B, D), lambda i: (i, 0)),
        out_shape=jax.ShapeDtypeStruct((N, D), jnp.float32),
    )(parts)


def _softplus(x):
    return jnp.maximum(x, 0.0) + jnp.log1p(jnp.exp(-jnp.abs(x)))


_LB = 2048


def _lane_consts():
    lane = lax.broadcasted_iota(jnp.int32, (1, 128), 1)
    w = jnp.where(lane == 0, 10.0, jnp.where(lane < 11, 1.0, 0.0))
    sgn = jnp.where(lane == 0, -1.0, jnp.where(lane < 11, 1.0, 0.0))
    return w, sgn


def _packed_logits(a, other, neg_ref, extra, bd):
    prods = [a * other] + [a * neg_ref[j] for j in range(NEG)] + [a * extra]
    x = jnp.concatenate(prods, axis=1)
    return jnp.dot(x, bd, preferred_element_type=jnp.float32)


def _loss_edges_body(row_ref, col_ref, neg_ref, gd_ref, w_ref, bd_ref,
                     og_ref, oa_ref):
    row = row_ref[...]
    col = col_ref[...]
    gdir = gd_ref[...]
    wstd = jnp.broadcast_to(w_ref[...], row.shape)
    bd = bd_ref[...]
    w, sgn = _lane_consts()

    p1 = _packed_logits(row, col, neg_ref, wstd, bd)
    p2 = _packed_logits(gdir, col, neg_ref, gdir, bd)
    std = jax.nn.sigmoid(p1[:, 11:12])
    c = std / jnp.maximum(jnp.sqrt(p2[:, 11:12]), 1e-12)
    base = p1 * sgn
    aug = (p1 + c * p2) * sgn
    gae_sum = jnp.sum(w * _softplus(base))
    aug_sum = jnp.sum(w * _softplus(aug))

    @pl.when(pl.program_id(0) == 0)
    def _():
        og_ref[...] = jnp.zeros_like(og_ref)
        oa_ref[...] = jnp.zeros_like(oa_ref)

    og_ref[...] = og_ref[...] + gae_sum
    oa_ref[...] = oa_ref[...] + aug_sum


def _loss_edges(row_h, col_h, neg3, gdir_rows, wstd_t, bd):
    return pl.pallas_call(
        _loss_edges_body,
        grid=(P // _LB,),
        in_specs=[pl.BlockSpec((_LB, D), lambda i: (i, 0)),
                  pl.BlockSpec((_LB, D), lambda i: (i, 0)),
                  pl.BlockSpec((NEG, _LB, D), lambda i: (0, i, 0)),
                  pl.BlockSpec((_LB, D), lambda i: (i, 0)),
                  pl.BlockSpec((1, D), lambda i: (0, 0)),
                  pl.BlockSpec((12 * D, 128), lambda i: (0, 0))],
        out_specs=[pl.BlockSpec((1, 1), lambda i: (0, 0)),
                   pl.BlockSpec((1, 1), lambda i: (0, 0))],
        out_shape=[jax.ShapeDtypeStruct((1, 1), jnp.float32),
                   jax.ShapeDtypeStruct((1, 1), jnp.float32)],
    )(row_h, col_h, neg3, gdir_rows, wstd_t, bd)


def _loss_inst_body(gae_ref, gdir_ref, neg_ref, w_ref, bd_ref, o_ref):
    gae_h = gae_ref[...]
    gdir = gdir_ref[...]
    wstd = jnp.broadcast_to(w_ref[...], gae_h.shape)
    bd = bd_ref[...]
    w, sgn = _lane_consts()
    w = jnp.where(w > 0.0, 1.0, 0.0)

    p1 = _packed_logits(gae_h, gae_h, neg_ref, wstd, bd)
    p2 = _packed_logits(gdir, gae_h, neg_ref, gdir, bd)
    std = jax.nn.sigmoid(p1[:, 11:12])
    c = std / jnp.maximum(jnp.sqrt(p2[:, 11:12]), 1e-12)
    logits = ((p1 + c * p2) * (1.0 / TEMP)) * sgn
    acc = jnp.sum(w * _softplus(logits))

    @pl.when(pl.program_id(0) == 0)
    def _():
        o_ref[...] = jnp.zeros_like(o_ref)

    o_ref[...] = o_ref[...] + acc


def _loss_inst(gae_h, gdir, idn3, wstd_t, bd):
    return pl.pallas_call(
        _loss_inst_body,
        grid=(S // _LB,),
        in_specs=[pl.BlockSpec((_LB, D), lambda i: (i, 0)),
                  pl.BlockSpec((_LB, D), lambda i: (i, 0)),
                  pl.BlockSpec((NEG, _LB, D), lambda i: (0, i, 0)),
                  pl.BlockSpec((1, D), lambda i: (0, 0)),
                  pl.BlockSpec((12 * D, 128), lambda i: (0, 0))],
        out_specs=pl.BlockSpec((1, 1), lambda i: (0, 0)),
        out_shape=jax.ShapeDtypeStruct((1, 1), jnp.float32),
    )(gae_h, gdir, idn3, wstd_t, bd)


def kernel(x, edge_index, gradint_dir, select_index, id2id, new_rows, cols,
           negs, id_negs, W1, W2, W_std):
    src = edge_index[0]
    dst = edge_index[1]
    pad = PAD_E - NE
    pad_i = jnp.arange(pad, dtype=jnp.int32)
    src_p = jnp.concatenate([src, pad_i % N]).reshape(NW, ECH, EK)
    dst_p = jnp.concatenate([dst, TRASH + pad_i % (ACC_ROWS - N)]).reshape(NW, ECH, EK)

    pre1 = _mm(x, W1)
    parts1 = _spmm(pre1, src_p, dst_p)
    pre2 = _relu_mm(parts1, W2)
    parts2 = _spmm(pre2, src_p, dst_p)
    h = _relu_add(parts2)

    sel_i = select_index.reshape(NW, 1, K)
    col_i = cols.reshape(NW, SEG_CH[2], K)
    neg_i = negs.T.reshape(NW, SEG_CH[3], K)
    idn_i = id_negs.T.reshape(NW, SEG_CH[4], K)
    gd_i = new_rows.reshape(NW, GD_CH, K)

    wstd_t = W_std.reshape(1, D)
    bd = jnp.repeat(jnp.eye(12, 128, dtype=jnp.float32), D, axis=0)

    gae_h, idn_f = _gather1(h, sel_i, idn_i)
    idn3 = idn_f.reshape(NEG, S, D)
    l2 = _loss_inst(gae_h, gradint_dir, idn3, wstd_t, bd)

    row_h, col_h, neg_f, gdir_rows = _gather2(
        h, gradint_dir, gae_h, col_i, neg_i, gd_i)
    neg3 = neg_f.reshape(NEG, P, D)
    lg, la = _loss_edges(row_h, col_h, neg3, gdir_rows, wstd_t, bd)

    gae_loss = lg[0, 0] / P
    aug_loss = la[0, 0] / P
    inst_loss = l2[0, 0] / S
    return gae_loss + AUG_W * aug_loss + INS_W * inst_loss
